# Initial kernel scaffold; baseline (speedup 1.0000x reference)
import jax
import jax.numpy as jnp
from jax.experimental import pallas as pl


def kernel(x1, edge_index1, e1, u1, x2, edge_index2, e2, u2, params):
    raise NotImplementedError("write your pallas kernel here")


# speedup vs baseline: 1.3202x; 1.3202x over previous
import functools

import jax
import jax.numpy as jnp
from jax import lax
from jax.experimental import pallas as pl
from jax.experimental.pallas import tpu as pltpu
from jax.experimental.pallas import tpu_sc as plsc

N = 10000
E = 320000
H = 128
F32 = jnp.float32


def _mlp3_call(segs, extra, bias_pg, W2, b2, W3, b3, *, block, bpg,
               gate=None, want_full=True, want_gmean=False, want_pmean=False,
               mean_scale=1.0):
    R = (extra if extra is not None else segs[0][0]).shape[0]
    grid = R // block
    assert R % block == 0
    ngr = bias_pg.shape[0]
    dout = W3.shape[1]

    inputs = []
    in_specs = []
    for (a, W) in segs:
        d = a.shape[1]
        inputs.append(a)
        in_specs.append(pl.BlockSpec((block, d), lambda i: (i, 0)))
        inputs.append(W)
        in_specs.append(pl.BlockSpec(W.shape, lambda i: (0, 0)))
    if extra is not None:
        inputs.append(extra)
        in_specs.append(pl.BlockSpec((block, 128), lambda i: (i, 0)))
    if gate is not None:
        inputs.append(gate)
        in_specs.append(pl.BlockSpec((block, dout), lambda i: (i, 0)))
    inputs += [bias_pg.reshape(ngr, 1, 128), W2, b2, W3, b3]
    in_specs += [
        pl.BlockSpec((1, 1, 128), lambda i: (i // bpg, 0, 0)),
        pl.BlockSpec(W2.shape, lambda i: (0, 0)),
        pl.BlockSpec((1, 128), lambda i: (0, 0)),
        pl.BlockSpec(W3.shape, lambda i: (0, 0)),
        pl.BlockSpec((1, dout), lambda i: (0, 0)),
    ]

    out_shape = []
    out_specs = []
    if want_full:
        out_shape.append(jax.ShapeDtypeStruct((R, dout), F32))
        out_specs.append(pl.BlockSpec((block, dout), lambda i: (i, 0)))
    if want_gmean:
        out_shape.append(jax.ShapeDtypeStruct((ngr, 1, dout), F32))
        out_specs.append(pl.BlockSpec((1, 1, dout), lambda i: (i // bpg, 0, 0)))
    if want_pmean:
        out_shape.append(jax.ShapeDtypeStruct((ngr, 1, dout), F32))
        out_specs.append(pl.BlockSpec((1, 1, dout), lambda i: (i // bpg, 0, 0)))

    nsegs = len(segs)
    has_extra = extra is not None
    has_gate = gate is not None

    def body(*refs):
        k = 0
        seg_refs = []
        for _ in range(nsegs):
            seg_refs.append((refs[k], refs[k + 1]))
            k += 2
        extra_ref = refs[k] if has_extra else None
        k += 1 if has_extra else 0
        gate_ref = refs[k] if has_gate else None
        k += 1 if has_gate else 0
        b1_ref, W2_ref, b2_ref, W3_ref, b3_ref = refs[k:k + 5]
        k += 5
        o_ref = refs[k] if want_full else None
        k += 1 if want_full else 0
        gm_ref = refs[k] if want_gmean else None
        k += 1 if want_gmean else 0
        pm_ref = refs[k] if want_pmean else None

        i = pl.program_id(0)
        h = jnp.broadcast_to(b1_ref[0], (block, 128))
        for (a_ref, w_ref) in seg_refs:
            h = h + jnp.dot(a_ref[...], w_ref[...], preferred_element_type=F32)
        if has_extra:
            h = h + extra_ref[...]
        h = jnp.maximum(h, 0.0)
        h = jnp.maximum(jnp.dot(h, W2_ref[...], preferred_element_type=F32)
                        + b2_ref[...], 0.0)
        o = jnp.dot(h, W3_ref[...], preferred_element_type=F32) + b3_ref[...]
        if want_full:
            o_ref[...] = o
        if want_gmean:
            v = jnp.sum(o * gate_ref[...], axis=0, keepdims=True)[None]

            @pl.when(i % bpg == 0)
            def _():
                gm_ref[...] = jnp.zeros_like(gm_ref)
            gm_ref[...] += v

            @pl.when(i % bpg == bpg - 1)
            def _():
                gm_ref[...] *= mean_scale
        if want_pmean:
            v2 = jnp.sum(o, axis=0, keepdims=True)[None]

            @pl.when(i % bpg == 0)
            def _():
                pm_ref[...] = jnp.zeros_like(pm_ref)
            pm_ref[...] += v2

            @pl.when(i % bpg == bpg - 1)
            def _():
                pm_ref[...] *= mean_scale

    res = pl.pallas_call(
        body, grid=(grid,), in_specs=in_specs,
        out_specs=out_specs if len(out_specs) > 1 else out_specs[0],
        out_shape=out_shape if len(out_shape) > 1 else out_shape[0],
    )(*inputs)
    res = list(res) if isinstance(res, (tuple, list)) else [res]
    k = 1 if want_full else 0
    for j in range(k, len(res)):
        res[j] = res[j].reshape(ngr, dout)
    return tuple(res)


def _proj_call(segs, *, block):
    R = segs[0][0].shape[0]
    grid = R // block
    inputs = []
    in_specs = []
    for (a, W) in segs:
        d = a.shape[1]
        inputs.append(a)
        in_specs.append(pl.BlockSpec((block, d), lambda i: (i, 0)))
        inputs.append(W)
        in_specs.append(pl.BlockSpec(W.shape, lambda i: (0, 0)))
    nsegs = len(segs)

    def body(*refs):
        acc = None
        for s in range(nsegs):
            a_ref, w_ref = refs[2 * s], refs[2 * s + 1]
            d = jnp.dot(a_ref[...], w_ref[...], preferred_element_type=F32)
            acc = d if acc is None else acc + d
        refs[-1][...] = acc

    return pl.pallas_call(
        body, grid=(grid,), in_specs=in_specs,
        out_specs=pl.BlockSpec((block, 128), lambda i: (i, 0)),
        out_shape=jax.ShapeDtypeStruct((R, 128), F32),
    )(*inputs)


def _tiny3(x, l1, l2, l3, gate_pair=None):
    r = x.shape[0]
    xp = jnp.pad(x, ((0, 8 - r), (0, 0)))
    (W1, b1), (W2, b2), (W3, b3) = l1, l2, l3
    dout = W3.shape[1]
    inputs = [xp, W1, b1.reshape(1, -1), W2, b2.reshape(1, -1),
              W3, b3.reshape(1, -1)]
    if gate_pair is not None:
        ga = jnp.pad(gate_pair[0], ((0, 8 - r), (0, 0)))
        gb = jnp.pad(gate_pair[1], ((0, 8 - r), (0, 0)))
        inputs += [ga, gb]
    has_gate = gate_pair is not None

    def body(*refs):
        x_ref, W1r, b1r, W2r, b2r, W3r, b3r = refs[:7]
        o_ref = refs[-1]
        xv = x_ref[...]
        if has_gate:
            xv = jnp.concatenate([xv, refs[7][...] * refs[8][...]], axis=1)
        h = jnp.maximum(jnp.dot(xv, W1r[...], preferred_element_type=F32)
                        + b1r[...], 0.0)
        h = jnp.maximum(jnp.dot(h, W2r[...], preferred_element_type=F32)
                        + b2r[...], 0.0)
        o_ref[...] = jnp.dot(h, W3r[...], preferred_element_type=F32) + b3r[...]

    out = pl.pallas_call(
        body, out_shape=jax.ShapeDtypeStruct((8, dout), F32),
    )(*inputs)
    return out[:r]


def _tiny_affine(x, W, b):
    r = x.shape[0]
    xp = jnp.pad(x, ((0, 8 - r), (0, 0)))

    def body(x_ref, w_ref, b_ref, o_ref):
        o_ref[...] = (jnp.dot(x_ref[...], w_ref[...], preferred_element_type=F32)
                      + b_ref[...])

    out = pl.pallas_call(
        body, out_shape=jax.ShapeDtypeStruct((8, W.shape[1]), F32),
    )(xp, W, b.reshape(1, -1))
    return out[:r]


_BQ = 256
_BK = 256
_NPAD = 10240


def _flash_cosine(qr, qh, sr, sh):
    grid_i = _NPAD // _BQ
    grid_j = _NPAD // _BK

    def body(qr_ref, qh_ref, sr_ref, sh_ref, o_ref, acc_ref, m_ref, l_ref):
        j = pl.program_id(1)

        @pl.when(j == 0)
        def _():
            acc_ref[...] = jnp.zeros_like(acc_ref)
            m_ref[...] = jnp.full_like(m_ref, -1e30)
            l_ref[...] = jnp.zeros_like(l_ref)

        q = jnp.concatenate([qr_ref[...], qh_ref[...]], axis=1)
        s = jnp.concatenate([sr_ref[...], sh_ref[...]], axis=1)
        qn = q / (jnp.sqrt(jnp.sum(q * q, axis=1, keepdims=True)) + 1e-8)
        sn = s / (jnp.sqrt(jnp.sum(s * s, axis=1, keepdims=True)) + 1e-8)
        logits = lax.dot_general(qn, sn, (((1,), (1,)), ((), ())),
                                 preferred_element_type=F32)
        col = j * _BK + lax.broadcasted_iota(jnp.int32, (_BQ, _BK), 1)
        logits = jnp.where(col < N, logits, -1e30)
        m_old = m_ref[...]
        m_new = jnp.maximum(m_old, jnp.max(logits, axis=1, keepdims=True))
        alpha = jnp.exp(m_old - m_new)
        p = jnp.exp(logits - m_new)
        l_ref[...] = l_ref[...] * alpha + jnp.sum(p, axis=1, keepdims=True)
        acc_ref[...] = (acc_ref[...] * alpha
                        + jnp.dot(p, s, preferred_element_type=F32))
        m_ref[...] = m_new

        @pl.when(j == grid_j - 1)
        def _():
            o_ref[...] = acc_ref[...] / l_ref[...]

    out = pl.pallas_call(
        body,
        grid=(grid_i, grid_j),
        in_specs=[
            pl.BlockSpec((_BQ, 128), lambda i, j: (i, 0)),
            pl.BlockSpec((_BQ, 128), lambda i, j: (i, 0)),
            pl.BlockSpec((_BK, 128), lambda i, j: (j, 0)),
            pl.BlockSpec((_BK, 128), lambda i, j: (j, 0)),
        ],
        out_specs=pl.BlockSpec((_BQ, 256), lambda i, j: (i, 0)),
        out_shape=jax.ShapeDtypeStruct((_NPAD, 256), F32),
        scratch_shapes=[
            pltpu.VMEM((_BQ, 256), F32),
            pltpu.VMEM((_BQ, 1), F32),
            pltpu.VMEM((_BQ, 1), F32),
        ],
    )(qr, qh, sr, sh)
    return out[:N]


def _pad_rows(a):
    return jnp.pad(a, ((0, _NPAD - a.shape[0]), (0, 0)))


def _sc_gather_diff(table, dstg, srcg):
    M = dstg.shape[0]
    NW, CH = 32, 80
    per_w = M // NW
    n_ch = per_w // CH
    mesh = plsc.VectorSubcoreMesh(core_axis_name="c", subcore_axis_name="s")

    @functools.partial(
        pl.kernel, mesh=mesh,
        out_type=jax.ShapeDtypeStruct((M, 128), F32),
        scratch_types=[
            pltpu.VMEM((CH,), jnp.int32),
            pltpu.VMEM((CH,), jnp.int32),
            pltpu.VMEM((CH, 128), F32),
            pltpu.VMEM((CH, 128), F32),
            pltpu.SemaphoreType.DMA,
            pltpu.SemaphoreType.DMA,
        ])
    def k(table_h, dst_h, src_h, out_h, idxd, idxs, bufd, bufs, semd, sems):
        wid = lax.axis_index("s") * 2 + lax.axis_index("c")
        base = wid * per_w

        def chunk(ci, carry):
            off = base + ci * CH
            pltpu.sync_copy(dst_h.at[pl.ds(off, CH)], idxd)
            pltpu.sync_copy(src_h.at[pl.ds(off, CH)], idxs)
            cpd = pltpu.async_copy(table_h.at[idxd], bufd, semd)
            cps = pltpu.async_copy(table_h.at[idxs], bufs, sems)
            cpd.wait()
            cps.wait()

            def row(r, c2):
                for l in range(8):
                    sl = pl.ds(l * 16, 16)
                    bufd[r, sl] = bufd[r, sl] - bufs[r, sl]
                return c2
            lax.fori_loop(0, CH, row, 0)
            pltpu.sync_copy(bufd, out_h.at[pl.ds(off, CH)])
            return carry
        lax.fori_loop(0, n_ch, chunk, 0)

    return k(table, dstg, srcg)


def _sc_scatter_add(vals, dst_local, zeros):
    per_t = E // 16
    CH = 80
    n_ch = per_t // CH
    rows_t = _NPAD // 16
    mesh = plsc.VectorSubcoreMesh(core_axis_name="c", subcore_axis_name="s")

    @functools.partial(
        pl.kernel, mesh=mesh,
        out_type=jax.ShapeDtypeStruct((2, _NPAD, 128), F32),
        scratch_types=[
            pltpu.VMEM((CH,), jnp.int32),
            pltpu.VMEM((CH, 128), F32),
            pltpu.VMEM_SHARED((_NPAD, 128), F32),
        ])
    def k(vals_h, dst_h, zeros_h, out_h, idxv, vbuf, acc):
        c = lax.axis_index("c")
        s = lax.axis_index("s")
        for j in range(rows_t // CH):
            sl = pl.ds(s * rows_t + j * CH, CH)
            pltpu.sync_copy(zeros_h.at[sl], vbuf)
            pltpu.sync_copy(vbuf, acc.at[sl])
        plsc.subcore_barrier()
        base = c * E + s * per_t

        def chunk(ci, carry):
            off = base + ci * CH
            pltpu.sync_copy(dst_h.at[pl.ds(off, CH)], idxv)
            pltpu.sync_copy(vals_h.at[pl.ds(off, CH)], vbuf)
            pltpu.sync_copy(vbuf, acc.at[idxv], add=True)
            return carry
        lax.fori_loop(0, n_ch, chunk, 0)
        plsc.subcore_barrier()
        for j in range(rows_t // CH):
            sl = pl.ds(s * rows_t + j * CH, CH)
            pltpu.sync_copy(acc.at[sl], vbuf)
            pltpu.sync_copy(vbuf, out_h.at[c, sl])

    return k(vals, dst_local, zeros)[:, :N]


def kernel(x1, edge_index1, e1, u1, x2, edge_index2, e2, u2, params):
    p = params

    x12 = jnp.concatenate([x1, x2], 0)
    e12 = jnp.concatenate([e1, e2], 0)
    src_l = jnp.concatenate([edge_index1[0], edge_index2[0]])
    dst_l = jnp.concatenate([edge_index1[1], edge_index2[1]])
    src_g = jnp.concatenate([edge_index1[0], edge_index2[0] + N])
    dst_g = jnp.concatenate([edge_index1[1], edge_index2[1] + N])
    zeros_n = jnp.zeros((_NPAD, 128), F32)

    (enW1, enb1), (enW2, enb2), (enW3, enb3) = p['enc_node']
    (xh12,) = _mlp3_call([(x12, enW1)], None, enb1.reshape(1, -1),
                         enW2, enb2.reshape(1, -1), enW3, enb3.reshape(1, -1),
                         block=800, bpg=25)
    (eeW1, eeb1), (eeW2, eeb2), (eeW3, eeb3) = p['enc_edge']
    (eh12,) = _mlp3_call([(e12, eeW1)], None, eeb1.reshape(1, -1),
                         eeW2, eeb2.reshape(1, -1), eeW3, eeb3.reshape(1, -1),
                         block=512, bpg=1250)
    uh = _tiny3(jnp.stack([u1, u2]), *p['enc_glob'])
    u_cat = jnp.concatenate([jnp.stack([u1, u2]), uh], 1)

    x1h, x2h = xh12[:N], xh12[N:]

    (W1, b1), (W2, b2), (W3, b3) = p['rec_edge']
    W1x, W1er, W1eh, W1u = W1[0:256], W1[256:272], W1[272:400], W1[400:656]
    Pt = _proj_call([(x12, W1x[:128]), (xh12, W1x[128:])], block=800)
    G = _sc_gather_diff(Pt, dst_g, src_g)
    bias_re = _tiny_affine(u_cat, W1u, b1)
    e_new, em = _mlp3_call(
        [(e12, W1er), (eh12, W1eh)], G, bias_re,
        W2, b2.reshape(1, -1), W3, b3.reshape(1, -1),
        block=512, bpg=625, want_pmean=True, mean_scale=1.0 / E)
    agg = _sc_scatter_add(e_new, dst_l, zeros_n)

    (N1, nb1), (N2, nb2), (N3, nb3) = p['rec_node']
    NWa, NWx, NWt, NWu = N1[0:128], N1[128:384], N1[384:640], N1[640:896]
    bias_rn = _tiny_affine(u_cat, NWu, nb1)

    att1 = _flash_cosine(_pad_rows(x1), _pad_rows(x1h),
                         _pad_rows(x2), _pad_rows(x2h))
    x1n, xm1 = _mlp3_call(
        [(agg[0], NWa), (x1, NWx[:128]), (x1h, NWx[128:]), (att1, NWt)],
        None, bias_rn[0:1], N2, nb2.reshape(1, -1), N3, nb3.reshape(1, -1),
        block=1000, bpg=10, want_pmean=True, mean_scale=1.0 / N)
    u1n = _tiny3(jnp.concatenate([xm1, em[0:1], u_cat[0:1]], 1), *p['rec_glob'])

    att2 = _flash_cosine(_pad_rows(x2), _pad_rows(x2h),
                         _pad_rows(x1), _pad_rows(x1n))
    x2n, xm2 = _mlp3_call(
        [(agg[1], NWa), (x2, NWx[:128]), (x2h, NWx[128:]), (att2, NWt)],
        None, bias_rn[1:2], N2, nb2.reshape(1, -1), N3, nb3.reshape(1, -1),
        block=1000, bpg=10, want_pmean=True, mean_scale=1.0 / N)
    u2n = _tiny3(jnp.concatenate([xm2, em[1:2], u_cat[1:2]], 1), *p['rec_glob'])

    un = jnp.concatenate([u1n, u2n], 0)
    xn = jnp.concatenate([x1n, x2n], 0)

    (A1, ab1), (A2, ab2), (A3, ab3) = p['att_edge']
    A1x, A1e, A1u = A1[0:128], A1[128:256], A1[256:384]
    Pa = _proj_call([(xn, A1x)], block=800)
    Ga = _sc_gather_diff(Pa, dst_g, src_g)
    bias_ae = _tiny_affine(un, A1u, ab1)
    ea, egm, eam = _mlp3_call(
        [(e_new, A1e)], Ga, bias_ae,
        A2, ab2.reshape(1, -1), A3, ab3.reshape(1, -1),
        block=512, bpg=625, gate=e_new, want_gmean=True, want_pmean=True,
        mean_scale=1.0 / E)
    agga = _sc_scatter_add(ea, dst_l, zeros_n)

    (B1, bb1), (B2, bb2), (B3, bb3) = p['att_node']
    B1a, B1x, B1u = B1[0:128], B1[128:256], B1[256:384]
    biasn = _tiny_affine(un, B1u, bb1)
    xgm, xam = _mlp3_call(
        [(agga.reshape(2 * N, 128), B1a), (xn, B1x)], None, biasn,
        B2, bb2.reshape(1, -1), B3, bb3.reshape(1, -1),
        block=1000, bpg=10, gate=xn, want_full=False, want_gmean=True,
        want_pmean=True, mean_scale=1.0 / N)

    ua = _tiny3(jnp.concatenate([xam, eam, un], 1), *p['att_glob'])
    uf = _tiny3(jnp.concatenate([xgm, egm], 1), *p['agg_glob'],
                gate_pair=(un, ua))
    out = _tiny3(uf.reshape(1, 256), *p['final'])
    return out.reshape(64)


# speedup vs baseline: 1.3424x; 1.0169x over previous
import functools

import jax
import jax.numpy as jnp
from jax import lax
from jax.experimental import pallas as pl
from jax.experimental.pallas import tpu as pltpu
from jax.experimental.pallas import tpu_sc as plsc

N = 10000
E = 320000
H = 128
F32 = jnp.float32


def _mlp3_call(segs, extra, bias_pg, W2, b2, W3, b3, *, block, bpg,
               gate=None, want_full=True, want_gmean=False, want_pmean=False,
               mean_scale=1.0):
    R = (extra if extra is not None else segs[0][0]).shape[0]
    grid = R // block
    assert R % block == 0
    ngr = bias_pg.shape[0]
    dout = W3.shape[1]

    inputs = []
    in_specs = []
    for (a, W) in segs:
        d = a.shape[1]
        inputs.append(a)
        in_specs.append(pl.BlockSpec((block, d), lambda i: (i, 0)))
        inputs.append(W)
        in_specs.append(pl.BlockSpec(W.shape, lambda i: (0, 0)))
    if extra is not None:
        inputs.append(extra)
        in_specs.append(pl.BlockSpec((block, 128), lambda i: (i, 0)))
    if gate is not None:
        inputs.append(gate)
        in_specs.append(pl.BlockSpec((block, dout), lambda i: (i, 0)))
    inputs += [bias_pg.reshape(ngr, 1, 128), W2, b2, W3, b3]
    in_specs += [
        pl.BlockSpec((1, 1, 128), lambda i: (i // bpg, 0, 0)),
        pl.BlockSpec(W2.shape, lambda i: (0, 0)),
        pl.BlockSpec((1, 128), lambda i: (0, 0)),
        pl.BlockSpec(W3.shape, lambda i: (0, 0)),
        pl.BlockSpec((1, dout), lambda i: (0, 0)),
    ]

    out_shape = []
    out_specs = []
    if want_full:
        out_shape.append(jax.ShapeDtypeStruct((R, dout), F32))
        out_specs.append(pl.BlockSpec((block, dout), lambda i: (i, 0)))
    if want_gmean:
        out_shape.append(jax.ShapeDtypeStruct((ngr, 1, dout), F32))
        out_specs.append(pl.BlockSpec((1, 1, dout), lambda i: (i // bpg, 0, 0)))
    if want_pmean:
        out_shape.append(jax.ShapeDtypeStruct((ngr, 1, dout), F32))
        out_specs.append(pl.BlockSpec((1, 1, dout), lambda i: (i // bpg, 0, 0)))

    nsegs = len(segs)
    has_extra = extra is not None
    has_gate = gate is not None

    def body(*refs):
        k = 0
        seg_refs = []
        for _ in range(nsegs):
            seg_refs.append((refs[k], refs[k + 1]))
            k += 2
        extra_ref = refs[k] if has_extra else None
        k += 1 if has_extra else 0
        gate_ref = refs[k] if has_gate else None
        k += 1 if has_gate else 0
        b1_ref, W2_ref, b2_ref, W3_ref, b3_ref = refs[k:k + 5]
        k += 5
        o_ref = refs[k] if want_full else None
        k += 1 if want_full else 0
        gm_ref = refs[k] if want_gmean else None
        k += 1 if want_gmean else 0
        pm_ref = refs[k] if want_pmean else None

        i = pl.program_id(0)
        h = jnp.broadcast_to(b1_ref[0], (block, 128))
        for (a_ref, w_ref) in seg_refs:
            h = h + jnp.dot(a_ref[...], w_ref[...], preferred_element_type=F32)
        if has_extra:
            h = h + extra_ref[...]
        h = jnp.maximum(h, 0.0)
        h = jnp.maximum(jnp.dot(h, W2_ref[...], preferred_element_type=F32)
                        + b2_ref[...], 0.0)
        o = jnp.dot(h, W3_ref[...], preferred_element_type=F32) + b3_ref[...]
        if want_full:
            o_ref[...] = o
        if want_gmean:
            v = jnp.sum(o * gate_ref[...], axis=0, keepdims=True)[None]

            @pl.when(i % bpg == 0)
            def _():
                gm_ref[...] = jnp.zeros_like(gm_ref)
            gm_ref[...] += v

            @pl.when(i % bpg == bpg - 1)
            def _():
                gm_ref[...] *= mean_scale
        if want_pmean:
            v2 = jnp.sum(o, axis=0, keepdims=True)[None]

            @pl.when(i % bpg == 0)
            def _():
                pm_ref[...] = jnp.zeros_like(pm_ref)
            pm_ref[...] += v2

            @pl.when(i % bpg == bpg - 1)
            def _():
                pm_ref[...] *= mean_scale

    res = pl.pallas_call(
        body, grid=(grid,), in_specs=in_specs,
        out_specs=out_specs if len(out_specs) > 1 else out_specs[0],
        out_shape=out_shape if len(out_shape) > 1 else out_shape[0],
    )(*inputs)
    res = list(res) if isinstance(res, (tuple, list)) else [res]
    k = 1 if want_full else 0
    for j in range(k, len(res)):
        res[j] = res[j].reshape(ngr, dout)
    return tuple(res)


def _proj_call(segs, *, block):
    R = segs[0][0].shape[0]
    grid = R // block
    inputs = []
    in_specs = []
    for (a, W) in segs:
        d = a.shape[1]
        inputs.append(a)
        in_specs.append(pl.BlockSpec((block, d), lambda i: (i, 0)))
        inputs.append(W)
        in_specs.append(pl.BlockSpec(W.shape, lambda i: (0, 0)))
    nsegs = len(segs)

    def body(*refs):
        acc = None
        for s in range(nsegs):
            a_ref, w_ref = refs[2 * s], refs[2 * s + 1]
            d = jnp.dot(a_ref[...], w_ref[...], preferred_element_type=F32)
            acc = d if acc is None else acc + d
        refs[-1][...] = acc

    return pl.pallas_call(
        body, grid=(grid,), in_specs=in_specs,
        out_specs=pl.BlockSpec((block, 128), lambda i: (i, 0)),
        out_shape=jax.ShapeDtypeStruct((R, 128), F32),
    )(*inputs)


def _tiny3(x, l1, l2, l3, gate_pair=None):
    r = x.shape[0]
    xp = jnp.pad(x, ((0, 8 - r), (0, 0)))
    (W1, b1), (W2, b2), (W3, b3) = l1, l2, l3
    dout = W3.shape[1]
    inputs = [xp, W1, b1.reshape(1, -1), W2, b2.reshape(1, -1),
              W3, b3.reshape(1, -1)]
    if gate_pair is not None:
        ga = jnp.pad(gate_pair[0], ((0, 8 - r), (0, 0)))
        gb = jnp.pad(gate_pair[1], ((0, 8 - r), (0, 0)))
        inputs += [ga, gb]
    has_gate = gate_pair is not None

    def body(*refs):
        x_ref, W1r, b1r, W2r, b2r, W3r, b3r = refs[:7]
        o_ref = refs[-1]
        xv = x_ref[...]
        if has_gate:
            xv = jnp.concatenate([xv, refs[7][...] * refs[8][...]], axis=1)
        h = jnp.maximum(jnp.dot(xv, W1r[...], preferred_element_type=F32)
                        + b1r[...], 0.0)
        h = jnp.maximum(jnp.dot(h, W2r[...], preferred_element_type=F32)
                        + b2r[...], 0.0)
        o_ref[...] = jnp.dot(h, W3r[...], preferred_element_type=F32) + b3r[...]

    out = pl.pallas_call(
        body, out_shape=jax.ShapeDtypeStruct((8, dout), F32),
    )(*inputs)
    return out[:r]


def _tiny_affine(x, W, b):
    r = x.shape[0]
    xp = jnp.pad(x, ((0, 8 - r), (0, 0)))

    def body(x_ref, w_ref, b_ref, o_ref):
        o_ref[...] = (jnp.dot(x_ref[...], w_ref[...], preferred_element_type=F32)
                      + b_ref[...])

    out = pl.pallas_call(
        body, out_shape=jax.ShapeDtypeStruct((8, W.shape[1]), F32),
    )(xp, W, b.reshape(1, -1))
    return out[:r]


_BQ = 256
_BK = 256
_NPAD = 10240


def _flash_cosine(qr, qh, sr, sh):
    grid_i = _NPAD // _BQ
    grid_j = _NPAD // _BK

    def body(qr_ref, qh_ref, sr_ref, sh_ref, o_ref, acc_ref, m_ref, l_ref):
        j = pl.program_id(1)

        @pl.when(j == 0)
        def _():
            acc_ref[...] = jnp.zeros_like(acc_ref)
            m_ref[...] = jnp.full_like(m_ref, -1e30)
            l_ref[...] = jnp.zeros_like(l_ref)

        q = jnp.concatenate([qr_ref[...], qh_ref[...]], axis=1)
        s = jnp.concatenate([sr_ref[...], sh_ref[...]], axis=1)
        qn = q / (jnp.sqrt(jnp.sum(q * q, axis=1, keepdims=True)) + 1e-8)
        sn = s / (jnp.sqrt(jnp.sum(s * s, axis=1, keepdims=True)) + 1e-8)
        logits = lax.dot_general(qn.astype(jnp.bfloat16), sn.astype(jnp.bfloat16),
                                 (((1,), (1,)), ((), ())),
                                 preferred_element_type=F32)
        col = j * _BK + lax.broadcasted_iota(jnp.int32, (_BQ, _BK), 1)
        logits = jnp.where(col < N, logits, -1e30)
        m_old = m_ref[...]
        m_new = jnp.maximum(m_old, jnp.max(logits, axis=1, keepdims=True))
        alpha = jnp.exp(m_old - m_new)
        p = jnp.exp(logits - m_new)
        l_ref[...] = l_ref[...] * alpha + jnp.sum(p, axis=1, keepdims=True)
        acc_ref[...] = (acc_ref[...] * alpha
                        + jnp.dot(p.astype(jnp.bfloat16), s.astype(jnp.bfloat16),
                                  preferred_element_type=F32))
        m_ref[...] = m_new

        @pl.when(j == grid_j - 1)
        def _():
            o_ref[...] = acc_ref[...] / l_ref[...]

    out = pl.pallas_call(
        body,
        grid=(grid_i, grid_j),
        in_specs=[
            pl.BlockSpec((_BQ, 128), lambda i, j: (i, 0)),
            pl.BlockSpec((_BQ, 128), lambda i, j: (i, 0)),
            pl.BlockSpec((_BK, 128), lambda i, j: (j, 0)),
            pl.BlockSpec((_BK, 128), lambda i, j: (j, 0)),
        ],
        out_specs=pl.BlockSpec((_BQ, 256), lambda i, j: (i, 0)),
        out_shape=jax.ShapeDtypeStruct((_NPAD, 256), F32),
        scratch_shapes=[
            pltpu.VMEM((_BQ, 256), F32),
            pltpu.VMEM((_BQ, 1), F32),
            pltpu.VMEM((_BQ, 1), F32),
        ],
    )(qr, qh, sr, sh)
    return out[:N]


def _pad_rows(a):
    return jnp.pad(a, ((0, _NPAD - a.shape[0]), (0, 0)))


def _sc_gather_diff(table_p, dstl, srcl):
    M = dstl.shape[0]
    CH = 80
    per_t = (M // 2) // 16
    n_ch = per_t // CH
    rows_t = _NPAD // 16
    mesh = plsc.VectorSubcoreMesh(core_axis_name="c", subcore_axis_name="s")

    @functools.partial(
        pl.kernel, mesh=mesh,
        out_type=jax.ShapeDtypeStruct((M, 128), F32),
        scratch_types=[
            pltpu.VMEM((CH,), jnp.int32),
            pltpu.VMEM((CH,), jnp.int32),
            pltpu.VMEM((CH, 128), F32),
            pltpu.VMEM((CH, 128), F32),
            pltpu.VMEM_SHARED((_NPAD, 128), F32),
            pltpu.SemaphoreType.DMA,
            pltpu.SemaphoreType.DMA,
        ])
    def k(table_h, dst_h, src_h, out_h, idxd, idxs, bufd, bufs, spm, semd, sems):
        c = lax.axis_index("c")
        s = lax.axis_index("s")
        for j in range(rows_t // CH):
            pltpu.sync_copy(
                table_h.at[pl.ds(c * _NPAD + s * rows_t + j * CH, CH)], bufd)
            pltpu.sync_copy(bufd, spm.at[pl.ds(s * rows_t + j * CH, CH)])
        plsc.subcore_barrier()
        base = c * (M // 2) + s * per_t

        def chunk(ci, carry):
            off = base + ci * CH
            pltpu.sync_copy(dst_h.at[pl.ds(off, CH)], idxd)
            pltpu.sync_copy(src_h.at[pl.ds(off, CH)], idxs)
            cpd = pltpu.async_copy(spm.at[idxd], bufd, semd)
            cps = pltpu.async_copy(spm.at[idxs], bufs, sems)
            cpd.wait()
            cps.wait()

            def row(r, c2):
                for l in range(8):
                    sl = pl.ds(l * 16, 16)
                    bufd[r, sl] = bufd[r, sl] - bufs[r, sl]
                return c2
            lax.fori_loop(0, CH, row, 0)
            pltpu.sync_copy(bufd, out_h.at[pl.ds(off, CH)])
            return carry
        lax.fori_loop(0, n_ch, chunk, 0)

    return k(table_p, dstl, srcl)


def _sc_scatter_add(vals, dst_local, zeros):
    per_t = E // 16
    CH = 80
    n_ch = per_t // CH
    rows_t = _NPAD // 16
    mesh = plsc.VectorSubcoreMesh(core_axis_name="c", subcore_axis_name="s")

    @functools.partial(
        pl.kernel, mesh=mesh,
        out_type=jax.ShapeDtypeStruct((2, _NPAD, 128), F32),
        scratch_types=[
            pltpu.VMEM((CH,), jnp.int32),
            pltpu.VMEM((CH, 128), F32),
            pltpu.VMEM_SHARED((_NPAD, 128), F32),
        ])
    def k(vals_h, dst_h, zeros_h, out_h, idxv, vbuf, acc):
        c = lax.axis_index("c")
        s = lax.axis_index("s")
        for j in range(rows_t // CH):
            sl = pl.ds(s * rows_t + j * CH, CH)
            pltpu.sync_copy(zeros_h.at[sl], vbuf)
            pltpu.sync_copy(vbuf, acc.at[sl])
        plsc.subcore_barrier()
        base = c * E + s * per_t

        def chunk(ci, carry):
            off = base + ci * CH
            pltpu.sync_copy(dst_h.at[pl.ds(off, CH)], idxv)
            pltpu.sync_copy(vals_h.at[pl.ds(off, CH)], vbuf)
            pltpu.sync_copy(vbuf, acc.at[idxv], add=True)
            return carry
        lax.fori_loop(0, n_ch, chunk, 0)
        plsc.subcore_barrier()
        for j in range(rows_t // CH):
            sl = pl.ds(s * rows_t + j * CH, CH)
            pltpu.sync_copy(acc.at[sl], vbuf)
            pltpu.sync_copy(vbuf, out_h.at[c, sl])

    return k(vals, dst_local, zeros)[:, :N]


def kernel(x1, edge_index1, e1, u1, x2, edge_index2, e2, u2, params):
    p = params

    x12 = jnp.concatenate([x1, x2], 0)
    e12 = jnp.concatenate([e1, e2], 0)
    src_l = jnp.concatenate([edge_index1[0], edge_index2[0]])
    dst_l = jnp.concatenate([edge_index1[1], edge_index2[1]])
    zeros_n = jnp.zeros((_NPAD, 128), F32)
    zpad = jnp.zeros((_NPAD - N, 128), F32)

    def _pad_table(t):
        return jnp.concatenate([t[:N], zpad, t[N:], zpad], 0)

    (enW1, enb1), (enW2, enb2), (enW3, enb3) = p['enc_node']
    (xh12,) = _mlp3_call([(x12, enW1)], None, enb1.reshape(1, -1),
                         enW2, enb2.reshape(1, -1), enW3, enb3.reshape(1, -1),
                         block=800, bpg=25)
    (eeW1, eeb1), (eeW2, eeb2), (eeW3, eeb3) = p['enc_edge']
    (eh12,) = _mlp3_call([(e12, eeW1)], None, eeb1.reshape(1, -1),
                         eeW2, eeb2.reshape(1, -1), eeW3, eeb3.reshape(1, -1),
                         block=512, bpg=1250)
    uh = _tiny3(jnp.stack([u1, u2]), *p['enc_glob'])
    u_cat = jnp.concatenate([jnp.stack([u1, u2]), uh], 1)

    x1h, x2h = xh12[:N], xh12[N:]

    (W1, b1), (W2, b2), (W3, b3) = p['rec_edge']
    W1x, W1er, W1eh, W1u = W1[0:256], W1[256:272], W1[272:400], W1[400:656]
    Pt = _proj_call([(x12, W1x[:128]), (xh12, W1x[128:])], block=800)
    G = _sc_gather_diff(_pad_table(Pt), dst_l, src_l)
    bias_re = _tiny_affine(u_cat, W1u, b1)
    e_new, em = _mlp3_call(
        [(e12, W1er), (eh12, W1eh)], G, bias_re,
        W2, b2.reshape(1, -1), W3, b3.reshape(1, -1),
        block=512, bpg=625, want_pmean=True, mean_scale=1.0 / E)
    agg = _sc_scatter_add(e_new, dst_l, zeros_n)

    (N1, nb1), (N2, nb2), (N3, nb3) = p['rec_node']
    NWa, NWx, NWt, NWu = N1[0:128], N1[128:384], N1[384:640], N1[640:896]
    bias_rn = _tiny_affine(u_cat, NWu, nb1)

    att1 = _flash_cosine(_pad_rows(x1), _pad_rows(x1h),
                         _pad_rows(x2), _pad_rows(x2h))
    x1n, xm1 = _mlp3_call(
        [(agg[0], NWa), (x1, NWx[:128]), (x1h, NWx[128:]), (att1, NWt)],
        None, bias_rn[0:1], N2, nb2.reshape(1, -1), N3, nb3.reshape(1, -1),
        block=1000, bpg=10, want_pmean=True, mean_scale=1.0 / N)
    u1n = _tiny3(jnp.concatenate([xm1, em[0:1], u_cat[0:1]], 1), *p['rec_glob'])

    att2 = _flash_cosine(_pad_rows(x2), _pad_rows(x2h),
                         _pad_rows(x1), _pad_rows(x1n))
    x2n, xm2 = _mlp3_call(
        [(agg[1], NWa), (x2, NWx[:128]), (x2h, NWx[128:]), (att2, NWt)],
        None, bias_rn[1:2], N2, nb2.reshape(1, -1), N3, nb3.reshape(1, -1),
        block=1000, bpg=10, want_pmean=True, mean_scale=1.0 / N)
    u2n = _tiny3(jnp.concatenate([xm2, em[1:2], u_cat[1:2]], 1), *p['rec_glob'])

    un = jnp.concatenate([u1n, u2n], 0)
    xn = jnp.concatenate([x1n, x2n], 0)

    (A1, ab1), (A2, ab2), (A3, ab3) = p['att_edge']
    A1x, A1e, A1u = A1[0:128], A1[128:256], A1[256:384]
    Pa = _proj_call([(xn, A1x)], block=800)
    Ga = _sc_gather_diff(_pad_table(Pa), dst_l, src_l)
    bias_ae = _tiny_affine(un, A1u, ab1)
    ea, egm, eam = _mlp3_call(
        [(e_new, A1e)], Ga, bias_ae,
        A2, ab2.reshape(1, -1), A3, ab3.reshape(1, -1),
        block=512, bpg=625, gate=e_new, want_gmean=True, want_pmean=True,
        mean_scale=1.0 / E)
    agga = _sc_scatter_add(ea, dst_l, zeros_n)

    (B1, bb1), (B2, bb2), (B3, bb3) = p['att_node']
    B1a, B1x, B1u = B1[0:128], B1[128:256], B1[256:384]
    biasn = _tiny_affine(un, B1u, bb1)
    xgm, xam = _mlp3_call(
        [(agga.reshape(2 * N, 128), B1a), (xn, B1x)], None, biasn,
        B2, bb2.reshape(1, -1), B3, bb3.reshape(1, -1),
        block=1000, bpg=10, gate=xn, want_full=False, want_gmean=True,
        want_pmean=True, mean_scale=1.0 / N)

    ua = _tiny3(jnp.concatenate([xam, eam, un], 1), *p['att_glob'])
    uf = _tiny3(jnp.concatenate([xgm, egm], 1), *p['agg_glob'],
                gate_pair=(un, ua))
    out = _tiny3(uf.reshape(1, 256), *p['final'])
    return out.reshape(64)


# speedup vs baseline: 2.4762x; 1.8446x over previous
import functools

import jax
import jax.numpy as jnp
from jax import lax
from jax.experimental import pallas as pl
from jax.experimental.pallas import tpu as pltpu
from jax.experimental.pallas import tpu_sc as plsc

N = 10000
E = 320000
H = 128
F32 = jnp.float32


def _mlp3_call(segs, extra, bias_pg, W2, b2, W3, b3, *, block, bpg,
               gate=None, want_full=True, want_gmean=False, want_pmean=False,
               mean_scale=1.0):
    R = (extra if extra is not None else segs[0][0]).shape[0]
    grid = R // block
    assert R % block == 0
    ngr = bias_pg.shape[0]
    dout = W3.shape[1]

    inputs = []
    in_specs = []
    for (a, W) in segs:
        d = a.shape[1]
        inputs.append(a)
        in_specs.append(pl.BlockSpec((block, d), lambda i: (i, 0)))
        inputs.append(W)
        in_specs.append(pl.BlockSpec(W.shape, lambda i: (0, 0)))
    if extra is not None:
        inputs.append(extra)
        in_specs.append(pl.BlockSpec((block, 128), lambda i: (i, 0)))
    if gate is not None:
        inputs.append(gate)
        in_specs.append(pl.BlockSpec((block, dout), lambda i: (i, 0)))
    inputs += [bias_pg.reshape(ngr, 1, 128), W2, b2, W3, b3]
    in_specs += [
        pl.BlockSpec((1, 1, 128), lambda i: (i // bpg, 0, 0)),
        pl.BlockSpec(W2.shape, lambda i: (0, 0)),
        pl.BlockSpec((1, 128), lambda i: (0, 0)),
        pl.BlockSpec(W3.shape, lambda i: (0, 0)),
        pl.BlockSpec((1, dout), lambda i: (0, 0)),
    ]

    out_shape = []
    out_specs = []
    if want_full:
        out_shape.append(jax.ShapeDtypeStruct((R, dout), F32))
        out_specs.append(pl.BlockSpec((block, dout), lambda i: (i, 0)))
    if want_gmean:
        out_shape.append(jax.ShapeDtypeStruct((ngr, 1, dout), F32))
        out_specs.append(pl.BlockSpec((1, 1, dout), lambda i: (i // bpg, 0, 0)))
    if want_pmean:
        out_shape.append(jax.ShapeDtypeStruct((ngr, 1, dout), F32))
        out_specs.append(pl.BlockSpec((1, 1, dout), lambda i: (i // bpg, 0, 0)))

    nsegs = len(segs)
    has_extra = extra is not None
    has_gate = gate is not None

    def body(*refs):
        k = 0
        seg_refs = []
        for _ in range(nsegs):
            seg_refs.append((refs[k], refs[k + 1]))
            k += 2
        extra_ref = refs[k] if has_extra else None
        k += 1 if has_extra else 0
        gate_ref = refs[k] if has_gate else None
        k += 1 if has_gate else 0
        b1_ref, W2_ref, b2_ref, W3_ref, b3_ref = refs[k:k + 5]
        k += 5
        o_ref = refs[k] if want_full else None
        k += 1 if want_full else 0
        gm_ref = refs[k] if want_gmean else None
        k += 1 if want_gmean else 0
        pm_ref = refs[k] if want_pmean else None

        i = pl.program_id(0)
        h = jnp.broadcast_to(b1_ref[0], (block, 128))
        for (a_ref, w_ref) in seg_refs:
            h = h + jnp.dot(a_ref[...], w_ref[...], preferred_element_type=F32)
        if has_extra:
            h = h + extra_ref[...]
        h = jnp.maximum(h, 0.0)
        h = jnp.maximum(jnp.dot(h, W2_ref[...], preferred_element_type=F32)
                        + b2_ref[...], 0.0)
        o = jnp.dot(h, W3_ref[...], preferred_element_type=F32) + b3_ref[...]
        if want_full:
            o_ref[...] = o
        if want_gmean:
            v = jnp.sum(o * gate_ref[...], axis=0, keepdims=True)[None]

            @pl.when(i % bpg == 0)
            def _():
                gm_ref[...] = jnp.zeros_like(gm_ref)
            gm_ref[...] += v

            @pl.when(i % bpg == bpg - 1)
            def _():
                gm_ref[...] *= mean_scale
        if want_pmean:
            v2 = jnp.sum(o, axis=0, keepdims=True)[None]

            @pl.when(i % bpg == 0)
            def _():
                pm_ref[...] = jnp.zeros_like(pm_ref)
            pm_ref[...] += v2

            @pl.when(i % bpg == bpg - 1)
            def _():
                pm_ref[...] *= mean_scale

    res = pl.pallas_call(
        body, grid=(grid,), in_specs=in_specs,
        out_specs=out_specs if len(out_specs) > 1 else out_specs[0],
        out_shape=out_shape if len(out_shape) > 1 else out_shape[0],
    )(*inputs)
    res = list(res) if isinstance(res, (tuple, list)) else [res]
    k = 1 if want_full else 0
    for j in range(k, len(res)):
        res[j] = res[j].reshape(ngr, dout)
    return tuple(res)


def _proj_call(segs, *, block):
    R = segs[0][0].shape[0]
    grid = R // block
    inputs = []
    in_specs = []
    for (a, W) in segs:
        d = a.shape[1]
        inputs.append(a)
        in_specs.append(pl.BlockSpec((block, d), lambda i: (i, 0)))
        inputs.append(W)
        in_specs.append(pl.BlockSpec(W.shape, lambda i: (0, 0)))
    nsegs = len(segs)

    def body(*refs):
        acc = None
        for s in range(nsegs):
            a_ref, w_ref = refs[2 * s], refs[2 * s + 1]
            d = jnp.dot(a_ref[...], w_ref[...], preferred_element_type=F32)
            acc = d if acc is None else acc + d
        refs[-1][...] = acc

    return pl.pallas_call(
        body, grid=(grid,), in_specs=in_specs,
        out_specs=pl.BlockSpec((block, 128), lambda i: (i, 0)),
        out_shape=jax.ShapeDtypeStruct((R, 128), F32),
    )(*inputs)


def _tiny3(x, l1, l2, l3, gate_pair=None):
    r = x.shape[0]
    xp = jnp.pad(x, ((0, 8 - r), (0, 0)))
    (W1, b1), (W2, b2), (W3, b3) = l1, l2, l3
    dout = W3.shape[1]
    inputs = [xp, W1, b1.reshape(1, -1), W2, b2.reshape(1, -1),
              W3, b3.reshape(1, -1)]
    if gate_pair is not None:
        ga = jnp.pad(gate_pair[0], ((0, 8 - r), (0, 0)))
        gb = jnp.pad(gate_pair[1], ((0, 8 - r), (0, 0)))
        inputs += [ga, gb]
    has_gate = gate_pair is not None

    def body(*refs):
        x_ref, W1r, b1r, W2r, b2r, W3r, b3r = refs[:7]
        o_ref = refs[-1]
        xv = x_ref[...]
        if has_gate:
            xv = jnp.concatenate([xv, refs[7][...] * refs[8][...]], axis=1)
        h = jnp.maximum(jnp.dot(xv, W1r[...], preferred_element_type=F32)
                        + b1r[...], 0.0)
        h = jnp.maximum(jnp.dot(h, W2r[...], preferred_element_type=F32)
                        + b2r[...], 0.0)
        o_ref[...] = jnp.dot(h, W3r[...], preferred_element_type=F32) + b3r[...]

    out = pl.pallas_call(
        body, out_shape=jax.ShapeDtypeStruct((8, dout), F32),
    )(*inputs)
    return out[:r]


def _tiny_affine(x, W, b):
    r = x.shape[0]
    xp = jnp.pad(x, ((0, 8 - r), (0, 0)))

    def body(x_ref, w_ref, b_ref, o_ref):
        o_ref[...] = (jnp.dot(x_ref[...], w_ref[...], preferred_element_type=F32)
                      + b_ref[...])

    out = pl.pallas_call(
        body, out_shape=jax.ShapeDtypeStruct((8, W.shape[1]), F32),
    )(xp, W, b.reshape(1, -1))
    return out[:r]


_BQ = 512
_BK = 1024
_NPAD = 10240


def _flash_cosine(qr, qh, sr, sh):
    grid_i = _NPAD // _BQ
    grid_j = _NPAD // _BK

    def body(qr_ref, qh_ref, sr_ref, sh_ref, o_ref, acc_ref, m_ref, l_ref):
        j = pl.program_id(1)

        @pl.when(j == 0)
        def _():
            acc_ref[...] = jnp.zeros_like(acc_ref)
            m_ref[...] = jnp.full_like(m_ref, -1e30)
            l_ref[...] = jnp.zeros_like(l_ref)

        q = jnp.concatenate([qr_ref[...], qh_ref[...]], axis=1)
        s = jnp.concatenate([sr_ref[...], sh_ref[...]], axis=1)
        qn = q / (jnp.sqrt(jnp.sum(q * q, axis=1, keepdims=True)) + 1e-8)
        sn = s / (jnp.sqrt(jnp.sum(s * s, axis=1, keepdims=True)) + 1e-8)
        logits = lax.dot_general(qn.astype(jnp.bfloat16), sn.astype(jnp.bfloat16),
                                 (((1,), (1,)), ((), ())),
                                 preferred_element_type=F32)
        col = j * _BK + lax.broadcasted_iota(jnp.int32, (_BQ, _BK), 1)
        logits = jnp.where(col < N, logits, -1e30)
        m_old = m_ref[...]
        m_new = jnp.maximum(m_old, jnp.max(logits, axis=1, keepdims=True))
        alpha = jnp.exp(m_old - m_new)
        p = jnp.exp(logits - m_new)
        l_ref[...] = l_ref[...] * alpha + jnp.sum(p, axis=1, keepdims=True)
        acc_ref[...] = (acc_ref[...] * alpha
                        + jnp.dot(p.astype(jnp.bfloat16), s.astype(jnp.bfloat16),
                                  preferred_element_type=F32))
        m_ref[...] = m_new

        @pl.when(j == grid_j - 1)
        def _():
            o_ref[...] = acc_ref[...] / l_ref[...]

    out = pl.pallas_call(
        body,
        grid=(grid_i, grid_j),
        in_specs=[
            pl.BlockSpec((_BQ, 128), lambda i, j: (i, 0)),
            pl.BlockSpec((_BQ, 128), lambda i, j: (i, 0)),
            pl.BlockSpec((_BK, 128), lambda i, j: (j, 0)),
            pl.BlockSpec((_BK, 128), lambda i, j: (j, 0)),
        ],
        out_specs=pl.BlockSpec((_BQ, 256), lambda i, j: (i, 0)),
        out_shape=jax.ShapeDtypeStruct((_NPAD, 256), F32),
        scratch_shapes=[
            pltpu.VMEM((_BQ, 256), F32),
            pltpu.VMEM((_BQ, 1), F32),
            pltpu.VMEM((_BQ, 1), F32),
        ],
    )(qr, qh, sr, sh)
    return out[:N]


def _pad_rows(a):
    return jnp.pad(a, ((0, _NPAD - a.shape[0]), (0, 0)))


def _sc_gather_diff(table_p, dstl, srcl):
    M = dstl.shape[0]
    CH = 80
    per_t = (M // 2) // 16
    n_ch = per_t // CH
    rows_t = _NPAD // 16
    mesh = plsc.VectorSubcoreMesh(core_axis_name="c", subcore_axis_name="s")

    @functools.partial(
        pl.kernel, mesh=mesh,
        out_type=jax.ShapeDtypeStruct((M, 128), F32),
        scratch_types=[
            pltpu.VMEM((CH,), jnp.int32),
            pltpu.VMEM((CH,), jnp.int32),
            pltpu.VMEM((CH, 128), F32),
            pltpu.VMEM((CH, 128), F32),
            pltpu.VMEM_SHARED((_NPAD, 128), F32),
            pltpu.SemaphoreType.DMA,
            pltpu.SemaphoreType.DMA,
        ])
    def k(table_h, dst_h, src_h, out_h, idxd, idxs, bufd, bufs, spm, semd, sems):
        c = lax.axis_index("c")
        s = lax.axis_index("s")
        for j in range(rows_t // CH):
            pltpu.sync_copy(
                table_h.at[pl.ds(c * _NPAD + s * rows_t + j * CH, CH)], bufd)
            pltpu.sync_copy(bufd, spm.at[pl.ds(s * rows_t + j * CH, CH)])
        plsc.subcore_barrier()
        base = c * (M // 2) + s * per_t

        def chunk(ci, carry):
            off = base + ci * CH
            pltpu.sync_copy(dst_h.at[pl.ds(off, CH)], idxd)
            pltpu.sync_copy(src_h.at[pl.ds(off, CH)], idxs)
            cpd = pltpu.async_copy(spm.at[idxd], bufd, semd)
            cps = pltpu.async_copy(spm.at[idxs], bufs, sems)
            cpd.wait()
            cps.wait()

            def row(r, c2):
                for l in range(8):
                    sl = pl.ds(l * 16, 16)
                    bufd[r, sl] = bufd[r, sl] - bufs[r, sl]
                return c2
            lax.fori_loop(0, CH, row, 0)
            pltpu.sync_copy(bufd, out_h.at[pl.ds(off, CH)])
            return carry
        lax.fori_loop(0, n_ch, chunk, 0)

    return k(table_p, dstl, srcl)


def _sc_scatter_add(vals, dst_local, zeros):
    per_t = E // 16
    CH = 80
    n_ch = per_t // CH
    rows_t = _NPAD // 16
    mesh = plsc.VectorSubcoreMesh(core_axis_name="c", subcore_axis_name="s")

    @functools.partial(
        pl.kernel, mesh=mesh,
        out_type=jax.ShapeDtypeStruct((2, _NPAD, 128), F32),
        scratch_types=[
            pltpu.VMEM((CH,), jnp.int32),
            pltpu.VMEM((CH, 128), F32),
            pltpu.VMEM_SHARED((_NPAD, 128), F32),
        ])
    def k(vals_h, dst_h, zeros_h, out_h, idxv, vbuf, acc):
        c = lax.axis_index("c")
        s = lax.axis_index("s")
        for j in range(rows_t // CH):
            sl = pl.ds(s * rows_t + j * CH, CH)
            pltpu.sync_copy(zeros_h.at[sl], vbuf)
            pltpu.sync_copy(vbuf, acc.at[sl])
        plsc.subcore_barrier()
        base = c * E + s * per_t

        def chunk(ci, carry):
            off = base + ci * CH
            pltpu.sync_copy(dst_h.at[pl.ds(off, CH)], idxv)
            pltpu.sync_copy(vals_h.at[pl.ds(off, CH)], vbuf)
            pltpu.sync_copy(vbuf, acc.at[idxv], add=True)
            return carry
        lax.fori_loop(0, n_ch, chunk, 0)
        plsc.subcore_barrier()
        for j in range(rows_t // CH):
            sl = pl.ds(s * rows_t + j * CH, CH)
            pltpu.sync_copy(acc.at[sl], vbuf)
            pltpu.sync_copy(vbuf, out_h.at[c, sl])

    return k(vals, dst_local, zeros)[:, :N]


def kernel(x1, edge_index1, e1, u1, x2, edge_index2, e2, u2, params):
    p = params

    x12 = jnp.concatenate([x1, x2], 0)
    e12 = jnp.concatenate([e1, e2], 0)
    src_l = jnp.concatenate([edge_index1[0], edge_index2[0]])
    dst_l = jnp.concatenate([edge_index1[1], edge_index2[1]])
    zeros_n = jnp.zeros((_NPAD, 128), F32)
    zpad = jnp.zeros((_NPAD - N, 128), F32)

    def _pad_table(t):
        return jnp.concatenate([t[:N], zpad, t[N:], zpad], 0)

    (enW1, enb1), (enW2, enb2), (enW3, enb3) = p['enc_node']
    (xh12,) = _mlp3_call([(x12, enW1)], None, enb1.reshape(1, -1),
                         enW2, enb2.reshape(1, -1), enW3, enb3.reshape(1, -1),
                         block=800, bpg=25)
    (eeW1, eeb1), (eeW2, eeb2), (eeW3, eeb3) = p['enc_edge']
    (eh12,) = _mlp3_call([(e12, eeW1)], None, eeb1.reshape(1, -1),
                         eeW2, eeb2.reshape(1, -1), eeW3, eeb3.reshape(1, -1),
                         block=2560, bpg=250)
    uh = _tiny3(jnp.stack([u1, u2]), *p['enc_glob'])
    u_cat = jnp.concatenate([jnp.stack([u1, u2]), uh], 1)

    x1h, x2h = xh12[:N], xh12[N:]

    (W1, b1), (W2, b2), (W3, b3) = p['rec_edge']
    W1x, W1er, W1eh, W1u = W1[0:256], W1[256:272], W1[272:400], W1[400:656]
    Pt = _proj_call([(x12, W1x[:128]), (xh12, W1x[128:])], block=800)
    G = _sc_gather_diff(_pad_table(Pt), dst_l, src_l)
    bias_re = _tiny_affine(u_cat, W1u, b1)
    e_new, em = _mlp3_call(
        [(e12, W1er), (eh12, W1eh)], G, bias_re,
        W2, b2.reshape(1, -1), W3, b3.reshape(1, -1),
        block=2560, bpg=125, want_pmean=True, mean_scale=1.0 / E)
    agg = _sc_scatter_add(e_new, dst_l, zeros_n)

    (N1, nb1), (N2, nb2), (N3, nb3) = p['rec_node']
    NWa, NWx, NWt, NWu = N1[0:128], N1[128:384], N1[384:640], N1[640:896]
    bias_rn = _tiny_affine(u_cat, NWu, nb1)

    att1 = _flash_cosine(_pad_rows(x1), _pad_rows(x1h),
                         _pad_rows(x2), _pad_rows(x2h))
    x1n, xm1 = _mlp3_call(
        [(agg[0], NWa), (x1, NWx[:128]), (x1h, NWx[128:]), (att1, NWt)],
        None, bias_rn[0:1], N2, nb2.reshape(1, -1), N3, nb3.reshape(1, -1),
        block=2000, bpg=5, want_pmean=True, mean_scale=1.0 / N)
    u1n = _tiny3(jnp.concatenate([xm1, em[0:1], u_cat[0:1]], 1), *p['rec_glob'])

    att2 = _flash_cosine(_pad_rows(x2), _pad_rows(x2h),
                         _pad_rows(x1), _pad_rows(x1n))
    x2n, xm2 = _mlp3_call(
        [(agg[1], NWa), (x2, NWx[:128]), (x2h, NWx[128:]), (att2, NWt)],
        None, bias_rn[1:2], N2, nb2.reshape(1, -1), N3, nb3.reshape(1, -1),
        block=2000, bpg=5, want_pmean=True, mean_scale=1.0 / N)
    u2n = _tiny3(jnp.concatenate([xm2, em[1:2], u_cat[1:2]], 1), *p['rec_glob'])

    un = jnp.concatenate([u1n, u2n], 0)
    xn = jnp.concatenate([x1n, x2n], 0)

    (A1, ab1), (A2, ab2), (A3, ab3) = p['att_edge']
    A1x, A1e, A1u = A1[0:128], A1[128:256], A1[256:384]
    Pa = _proj_call([(xn, A1x)], block=800)
    Ga = _sc_gather_diff(_pad_table(Pa), dst_l, src_l)
    bias_ae = _tiny_affine(un, A1u, ab1)
    ea, egm, eam = _mlp3_call(
        [(e_new, A1e)], Ga, bias_ae,
        A2, ab2.reshape(1, -1), A3, ab3.reshape(1, -1),
        block=2560, bpg=125, gate=e_new, want_gmean=True, want_pmean=True,
        mean_scale=1.0 / E)
    agga = _sc_scatter_add(ea, dst_l, zeros_n)

    (B1, bb1), (B2, bb2), (B3, bb3) = p['att_node']
    B1a, B1x, B1u = B1[0:128], B1[128:256], B1[256:384]
    biasn = _tiny_affine(un, B1u, bb1)
    xgm, xam = _mlp3_call(
        [(agga.reshape(2 * N, 128), B1a), (xn, B1x)], None, biasn,
        B2, bb2.reshape(1, -1), B3, bb3.reshape(1, -1),
        block=2000, bpg=5, gate=xn, want_full=False, want_gmean=True,
        want_pmean=True, mean_scale=1.0 / N)

    ua = _tiny3(jnp.concatenate([xam, eam, un], 1), *p['att_glob'])
    uf = _tiny3(jnp.concatenate([xgm, egm], 1), *p['agg_glob'],
                gate_pair=(un, ua))
    out = _tiny3(uf.reshape(1, 256), *p['final'])
    return out.reshape(64)


# speedup vs baseline: 2.5388x; 1.0253x over previous
import functools

import jax
import jax.numpy as jnp
from jax import lax
from jax.experimental import pallas as pl
from jax.experimental.pallas import tpu as pltpu
from jax.experimental.pallas import tpu_sc as plsc

N = 10000
E = 320000
H = 128
F32 = jnp.float32


def _mlp3_call(segs, extra, bias_pg, W2, b2, W3, b3, *, block, bpg,
               gate=None, want_full=True, want_gmean=False, want_pmean=False,
               mean_scale=1.0):
    R = (extra if extra is not None else segs[0][0]).shape[0]
    grid = R // block
    assert R % block == 0
    ngr = bias_pg.shape[0]
    dout = W3.shape[1]

    inputs = []
    in_specs = []
    for (a, W) in segs:
        d = a.shape[1]
        inputs.append(a)
        in_specs.append(pl.BlockSpec((block, d), lambda i: (i, 0)))
        inputs.append(W)
        in_specs.append(pl.BlockSpec(W.shape, lambda i: (0, 0)))
    if extra is not None:
        inputs.append(extra)
        in_specs.append(pl.BlockSpec((block, 128), lambda i: (i, 0)))
    if gate is not None:
        inputs.append(gate)
        in_specs.append(pl.BlockSpec((block, dout), lambda i: (i, 0)))
    inputs += [bias_pg.reshape(ngr, 1, 128), W2, b2, W3, b3]
    in_specs += [
        pl.BlockSpec((1, 1, 128), lambda i: (i // bpg, 0, 0)),
        pl.BlockSpec(W2.shape, lambda i: (0, 0)),
        pl.BlockSpec((1, 128), lambda i: (0, 0)),
        pl.BlockSpec(W3.shape, lambda i: (0, 0)),
        pl.BlockSpec((1, dout), lambda i: (0, 0)),
    ]

    out_shape = []
    out_specs = []
    if want_full:
        out_shape.append(jax.ShapeDtypeStruct((R, dout), F32))
        out_specs.append(pl.BlockSpec((block, dout), lambda i: (i, 0)))
    if want_gmean:
        out_shape.append(jax.ShapeDtypeStruct((ngr, 1, dout), F32))
        out_specs.append(pl.BlockSpec((1, 1, dout), lambda i: (i // bpg, 0, 0)))
    if want_pmean:
        out_shape.append(jax.ShapeDtypeStruct((ngr, 1, dout), F32))
        out_specs.append(pl.BlockSpec((1, 1, dout), lambda i: (i // bpg, 0, 0)))

    nsegs = len(segs)
    has_extra = extra is not None
    has_gate = gate is not None

    def body(*refs):
        k = 0
        seg_refs = []
        for _ in range(nsegs):
            seg_refs.append((refs[k], refs[k + 1]))
            k += 2
        extra_ref = refs[k] if has_extra else None
        k += 1 if has_extra else 0
        gate_ref = refs[k] if has_gate else None
        k += 1 if has_gate else 0
        b1_ref, W2_ref, b2_ref, W3_ref, b3_ref = refs[k:k + 5]
        k += 5
        o_ref = refs[k] if want_full else None
        k += 1 if want_full else 0
        gm_ref = refs[k] if want_gmean else None
        k += 1 if want_gmean else 0
        pm_ref = refs[k] if want_pmean else None

        i = pl.program_id(0)
        h = jnp.broadcast_to(b1_ref[0], (block, 128))
        for (a_ref, w_ref) in seg_refs:
            h = h + jnp.dot(a_ref[...], w_ref[...], preferred_element_type=F32)
        if has_extra:
            h = h + extra_ref[...]
        h = jnp.maximum(h, 0.0)
        h = jnp.maximum(jnp.dot(h, W2_ref[...], preferred_element_type=F32)
                        + b2_ref[...], 0.0)
        o = jnp.dot(h, W3_ref[...], preferred_element_type=F32) + b3_ref[...]
        if want_full:
            o_ref[...] = o
        if want_gmean:
            v = jnp.sum(o * gate_ref[...], axis=0, keepdims=True)[None]

            @pl.when(i % bpg == 0)
            def _():
                gm_ref[...] = jnp.zeros_like(gm_ref)
            gm_ref[...] += v

            @pl.when(i % bpg == bpg - 1)
            def _():
                gm_ref[...] *= mean_scale
        if want_pmean:
            v2 = jnp.sum(o, axis=0, keepdims=True)[None]

            @pl.when(i % bpg == 0)
            def _():
                pm_ref[...] = jnp.zeros_like(pm_ref)
            pm_ref[...] += v2

            @pl.when(i % bpg == bpg - 1)
            def _():
                pm_ref[...] *= mean_scale

    res = pl.pallas_call(
        body, grid=(grid,), in_specs=in_specs,
        out_specs=out_specs if len(out_specs) > 1 else out_specs[0],
        out_shape=out_shape if len(out_shape) > 1 else out_shape[0],
    )(*inputs)
    res = list(res) if isinstance(res, (tuple, list)) else [res]
    k = 1 if want_full else 0
    for j in range(k, len(res)):
        res[j] = res[j].reshape(ngr, dout)
    return tuple(res)


def _proj_call(segs, *, block):
    R = segs[0][0].shape[0]
    grid = R // block
    inputs = []
    in_specs = []
    for (a, W) in segs:
        d = a.shape[1]
        inputs.append(a)
        in_specs.append(pl.BlockSpec((block, d), lambda i: (i, 0)))
        inputs.append(W)
        in_specs.append(pl.BlockSpec(W.shape, lambda i: (0, 0)))
    nsegs = len(segs)

    def body(*refs):
        acc = None
        for s in range(nsegs):
            a_ref, w_ref = refs[2 * s], refs[2 * s + 1]
            d = jnp.dot(a_ref[...], w_ref[...], preferred_element_type=F32)
            acc = d if acc is None else acc + d
        refs[-1][...] = acc

    return pl.pallas_call(
        body, grid=(grid,), in_specs=in_specs,
        out_specs=pl.BlockSpec((block, 128), lambda i: (i, 0)),
        out_shape=jax.ShapeDtypeStruct((R, 128), F32),
    )(*inputs)


def _tiny3(x, l1, l2, l3, gate_pair=None):
    r = x.shape[0]
    xp = jnp.pad(x, ((0, 8 - r), (0, 0)))
    (W1, b1), (W2, b2), (W3, b3) = l1, l2, l3
    dout = W3.shape[1]
    inputs = [xp, W1, b1.reshape(1, -1), W2, b2.reshape(1, -1),
              W3, b3.reshape(1, -1)]
    if gate_pair is not None:
        ga = jnp.pad(gate_pair[0], ((0, 8 - r), (0, 0)))
        gb = jnp.pad(gate_pair[1], ((0, 8 - r), (0, 0)))
        inputs += [ga, gb]
    has_gate = gate_pair is not None

    def body(*refs):
        x_ref, W1r, b1r, W2r, b2r, W3r, b3r = refs[:7]
        o_ref = refs[-1]
        xv = x_ref[...]
        if has_gate:
            xv = jnp.concatenate([xv, refs[7][...] * refs[8][...]], axis=1)
        h = jnp.maximum(jnp.dot(xv, W1r[...], preferred_element_type=F32)
                        + b1r[...], 0.0)
        h = jnp.maximum(jnp.dot(h, W2r[...], preferred_element_type=F32)
                        + b2r[...], 0.0)
        o_ref[...] = jnp.dot(h, W3r[...], preferred_element_type=F32) + b3r[...]

    out = pl.pallas_call(
        body, out_shape=jax.ShapeDtypeStruct((8, dout), F32),
    )(*inputs)
    return out[:r]


def _tiny_affine(x, W, b):
    r = x.shape[0]
    xp = jnp.pad(x, ((0, 8 - r), (0, 0)))

    def body(x_ref, w_ref, b_ref, o_ref):
        o_ref[...] = (jnp.dot(x_ref[...], w_ref[...], preferred_element_type=F32)
                      + b_ref[...])

    out = pl.pallas_call(
        body, out_shape=jax.ShapeDtypeStruct((8, W.shape[1]), F32),
    )(xp, W, b.reshape(1, -1))
    return out[:r]


_BQ = 512
_BK = 1024
_NPAD = 10240


def _flash_cosine(qr, qh, sr, sh):
    grid_i = _NPAD // _BQ
    grid_j = _NPAD // _BK

    def body(qr_ref, qh_ref, sr_ref, sh_ref, o_ref, acc_ref, m_ref, l_ref):
        j = pl.program_id(1)

        @pl.when(j == 0)
        def _():
            acc_ref[...] = jnp.zeros_like(acc_ref)
            m_ref[...] = jnp.full_like(m_ref, -1e30)
            l_ref[...] = jnp.zeros_like(l_ref)

        q = jnp.concatenate([qr_ref[...], qh_ref[...]], axis=1)
        s = jnp.concatenate([sr_ref[...], sh_ref[...]], axis=1)
        qn = q / (jnp.sqrt(jnp.sum(q * q, axis=1, keepdims=True)) + 1e-8)
        sn = s / (jnp.sqrt(jnp.sum(s * s, axis=1, keepdims=True)) + 1e-8)
        logits = lax.dot_general(qn.astype(jnp.bfloat16), sn.astype(jnp.bfloat16),
                                 (((1,), (1,)), ((), ())),
                                 preferred_element_type=F32)
        col = j * _BK + lax.broadcasted_iota(jnp.int32, (_BQ, _BK), 1)
        logits = jnp.where(col < N, logits, -1e30)
        m_old = m_ref[...]
        m_new = jnp.maximum(m_old, jnp.max(logits, axis=1, keepdims=True))
        alpha = jnp.exp(m_old - m_new)
        p = jnp.exp(logits - m_new)
        l_ref[...] = l_ref[...] * alpha + jnp.sum(p, axis=1, keepdims=True)
        acc_ref[...] = (acc_ref[...] * alpha
                        + jnp.dot(p.astype(jnp.bfloat16), s.astype(jnp.bfloat16),
                                  preferred_element_type=F32))
        m_ref[...] = m_new

        @pl.when(j == grid_j - 1)
        def _():
            o_ref[...] = acc_ref[...] / l_ref[...]

    out = pl.pallas_call(
        body,
        grid=(grid_i, grid_j),
        in_specs=[
            pl.BlockSpec((_BQ, 128), lambda i, j: (i, 0)),
            pl.BlockSpec((_BQ, 128), lambda i, j: (i, 0)),
            pl.BlockSpec((_BK, 128), lambda i, j: (j, 0)),
            pl.BlockSpec((_BK, 128), lambda i, j: (j, 0)),
        ],
        out_specs=pl.BlockSpec((_BQ, 256), lambda i, j: (i, 0)),
        out_shape=jax.ShapeDtypeStruct((_NPAD, 256), F32),
        scratch_shapes=[
            pltpu.VMEM((_BQ, 256), F32),
            pltpu.VMEM((_BQ, 1), F32),
            pltpu.VMEM((_BQ, 1), F32),
        ],
    )(qr, qh, sr, sh)
    return out[:N]


def _pad_rows(a):
    return jnp.pad(a, ((0, _NPAD - a.shape[0]), (0, 0)))


def _sc_gather_diff(table_p, dstl, srcl):
    CH = 40
    n_ch = dstl.shape[1]
    per_t = n_ch * CH
    M = 32 * per_t
    rows_t = _NPAD // 16
    mesh = plsc.VectorSubcoreMesh(core_axis_name="c", subcore_axis_name="s")

    @functools.partial(
        pl.kernel, mesh=mesh,
        out_type=jax.ShapeDtypeStruct((M, 128), F32),
        scratch_types=[
            pltpu.VMEM((CH,), jnp.int32),
            pltpu.VMEM((CH,), jnp.int32),
            pltpu.VMEM((CH,), jnp.int32),
            pltpu.VMEM((CH,), jnp.int32),
            pltpu.VMEM((CH, 128), F32),
            pltpu.VMEM((CH, 128), F32),
            pltpu.VMEM((CH, 128), F32),
            pltpu.VMEM((CH, 128), F32),
            pltpu.VMEM_SHARED((_NPAD, 128), F32),
            pltpu.SemaphoreType.DMA,
            pltpu.SemaphoreType.DMA,
            pltpu.SemaphoreType.DMA,
            pltpu.SemaphoreType.DMA,
            pltpu.SemaphoreType.DMA,
            pltpu.SemaphoreType.DMA,
            pltpu.SemaphoreType.DMA,
            pltpu.SemaphoreType.DMA,
            pltpu.SemaphoreType.DMA,
            pltpu.SemaphoreType.DMA,
        ])
    def k(table_h, dst_h, src_h, out_h,
          idxd0, idxd1, idxs0, idxs1,
          bufd0, bufd1, bufs0, bufs1, spm,
          gd0, gd1, gs0, gs1, so0, so1, id0, id1, is0, is1):
        c = lax.axis_index("c")
        s = lax.axis_index("s")
        w = c * 16 + s
        for j in range(rows_t // CH):
            pltpu.sync_copy(
                table_h.at[pl.ds(c * _NPAD + s * rows_t + j * CH, CH)], bufd0)
            pltpu.sync_copy(bufd0, spm.at[pl.ds(s * rows_t + j * CH, CH)])
        plsc.subcore_barrier()
        base = c * (M // 2) + s * per_t

        idxd = (idxd0, idxd1)
        idxs = (idxs0, idxs1)
        bufd = (bufd0, bufd1)
        bufs = (bufs0, bufs1)
        gsd = (gd0, gd1)
        gss = (gs0, gs1)
        osem = (so0, so1)
        isd = (id0, id1)
        iss = (is0, is1)

        def idx_issue(ci, b):
            pltpu.async_copy(dst_h.at[w, ci], idxd[b], isd[b])
            pltpu.async_copy(src_h.at[w, ci], idxs[b], iss[b])

        def wait_i(b):
            pltpu.make_async_copy(dst_h.at[w, 0], idxd[b], isd[b]).wait()
            pltpu.make_async_copy(src_h.at[w, 0], idxs[b], iss[b]).wait()

        def issue(b):
            pltpu.async_copy(spm.at[idxd[b]], bufd[b], gsd[b])
            pltpu.async_copy(spm.at[idxs[b]], bufs[b], gss[b])

        def wait_g(b):
            pltpu.make_async_copy(spm.at[idxd[b]], bufd[b], gsd[b]).wait()
            pltpu.make_async_copy(spm.at[idxs[b]], bufs[b], gss[b]).wait()

        def sub(b):
            D, S = bufd[b], bufs[b]

            def row(r, c2):
                for l in range(8):
                    sl = pl.ds(l * 16, 16)
                    D[r, sl] = D[r, sl] - S[r, sl]
                return c2
            lax.fori_loop(0, CH, row, 0, unroll=8)

        def out_issue(ci, b):
            pltpu.async_copy(bufd[b], out_h.at[pl.ds(base + ci * CH, CH)],
                             osem[b])

        def wait_out(b):
            pltpu.make_async_copy(bufd[b], out_h.at[pl.ds(base, CH)],
                                  osem[b]).wait()

        idx_issue(0, 0)
        idx_issue(1, 1)
        wait_i(0)
        issue(0)
        wait_g(0); idx_issue(2, 0); wait_i(1); issue(1); sub(0)
        out_issue(0, 0)
        wait_g(1); idx_issue(3, 1); wait_out(0); wait_i(0); issue(0); sub(1)
        out_issue(1, 1)

        def body(t, carry):
            i0 = 2 * t
            wait_g(0); idx_issue(i0 + 2, 0); wait_out(1); wait_i(1); issue(1)
            sub(0); out_issue(i0, 0)
            wait_g(1); idx_issue(i0 + 3, 1); wait_out(0); wait_i(0); issue(0)
            sub(1); out_issue(i0 + 1, 1)
            return carry
        lax.fori_loop(1, n_ch // 2 - 1, body, 0)

        wait_g(0); wait_out(1); wait_i(1); issue(1); sub(0)
        out_issue(n_ch - 2, 0)
        wait_g(1); wait_out(0); sub(1); out_issue(n_ch - 1, 1)
        wait_out(1)

    return k(table_p, dstl, srcl)


def _sc_scatter_add(vals, dst_local, zeros):
    per_t = E // 16
    CH = 80
    n_ch = per_t // CH
    rows_t = _NPAD // 16
    mesh = plsc.VectorSubcoreMesh(core_axis_name="c", subcore_axis_name="s")

    @functools.partial(
        pl.kernel, mesh=mesh,
        out_type=jax.ShapeDtypeStruct((2, _NPAD, 128), F32),
        scratch_types=[
            pltpu.VMEM((CH,), jnp.int32),
            pltpu.VMEM((CH,), jnp.int32),
            pltpu.VMEM((CH, 128), F32),
            pltpu.VMEM((CH, 128), F32),
            pltpu.VMEM_SHARED((_NPAD, 128), F32),
            pltpu.SemaphoreType.DMA,
            pltpu.SemaphoreType.DMA,
            pltpu.SemaphoreType.DMA,
            pltpu.SemaphoreType.DMA,
        ])
    def k(vals_h, dst_h, zeros_h, out_h, idx0, idx1, vbuf0, vbuf1, acc,
          vs0, vs1, is0, is1):
        c = lax.axis_index("c")
        s = lax.axis_index("s")
        w = c * 16 + s
        for j in range(rows_t // CH):
            sl = pl.ds(s * rows_t + j * CH, CH)
            pltpu.sync_copy(zeros_h.at[sl], vbuf0)
            pltpu.sync_copy(vbuf0, acc.at[sl])
        plsc.subcore_barrier()
        base = c * E + s * per_t

        idxb = (idx0, idx1)
        isem = (is0, is1)
        vbuf = (vbuf0, vbuf1)
        vsem = (vs0, vs1)

        def idx_issue(ci, b):
            pltpu.async_copy(dst_h.at[w, ci], idxb[b], isem[b])

        def wait_i(b):
            pltpu.make_async_copy(dst_h.at[w, 0], idxb[b], isem[b]).wait()

        def load_issue(ci, b):
            pltpu.async_copy(vals_h.at[pl.ds(base + ci * CH, CH)], vbuf[b],
                             vsem[b])

        def wait_v(b):
            pltpu.make_async_copy(vals_h.at[pl.ds(base, CH)], vbuf[b],
                                  vsem[b]).wait()

        def scat(b):
            pltpu.sync_copy(vbuf[b], acc.at[idxb[b]], add=True)

        idx_issue(0, 0)
        idx_issue(1, 1)
        load_issue(0, 0)

        def body(t, carry):
            i0 = 2 * t
            load_issue(i0 + 1, 1); wait_v(0); wait_i(0); scat(0)
            idx_issue(i0 + 2, 0)
            load_issue(i0 + 2, 0); wait_v(1); wait_i(1); scat(1)
            idx_issue(i0 + 3, 1)
            return carry
        lax.fori_loop(0, n_ch // 2 - 1, body, 0)

        load_issue(n_ch - 1, 1); wait_v(0); wait_i(0); scat(0)
        wait_v(1); wait_i(1); scat(1)

        plsc.subcore_barrier()
        for j in range(rows_t // CH):
            sl = pl.ds(s * rows_t + j * CH, CH)
            pltpu.sync_copy(acc.at[sl], vbuf0)
            pltpu.sync_copy(vbuf0, out_h.at[c, sl])

    return k(vals, dst_local, zeros)[:, :N]


def kernel(x1, edge_index1, e1, u1, x2, edge_index2, e2, u2, params):
    p = params

    x12 = jnp.concatenate([x1, x2], 0)
    e12 = jnp.concatenate([e1, e2], 0)
    src_l = jnp.concatenate([edge_index1[0], edge_index2[0]])
    dst_l = jnp.concatenate([edge_index1[1], edge_index2[1]])
    src3 = src_l.reshape(32, 250, 80)
    dst3 = dst_l.reshape(32, 250, 80)
    src3g = src_l.reshape(32, 500, 40)
    dst3g = dst_l.reshape(32, 500, 40)
    zeros_n = jnp.zeros((_NPAD, 128), F32)
    zpad = jnp.zeros((_NPAD - N, 128), F32)

    def _pad_table(t):
        return jnp.concatenate([t[:N], zpad, t[N:], zpad], 0)

    (enW1, enb1), (enW2, enb2), (enW3, enb3) = p['enc_node']
    (xh12,) = _mlp3_call([(x12, enW1)], None, enb1.reshape(1, -1),
                         enW2, enb2.reshape(1, -1), enW3, enb3.reshape(1, -1),
                         block=800, bpg=25)
    (eeW1, eeb1), (eeW2, eeb2), (eeW3, eeb3) = p['enc_edge']
    (eh12,) = _mlp3_call([(e12, eeW1)], None, eeb1.reshape(1, -1),
                         eeW2, eeb2.reshape(1, -1), eeW3, eeb3.reshape(1, -1),
                         block=2560, bpg=250)
    uh = _tiny3(jnp.stack([u1, u2]), *p['enc_glob'])
    u_cat = jnp.concatenate([jnp.stack([u1, u2]), uh], 1)

    x1h, x2h = xh12[:N], xh12[N:]

    (W1, b1), (W2, b2), (W3, b3) = p['rec_edge']
    W1x, W1er, W1eh, W1u = W1[0:256], W1[256:272], W1[272:400], W1[400:656]
    Pt = _proj_call([(x12, W1x[:128]), (xh12, W1x[128:])], block=800)
    G = _sc_gather_diff(_pad_table(Pt), dst3g, src3g)
    bias_re = _tiny_affine(u_cat, W1u, b1)
    e_new, em = _mlp3_call(
        [(e12, W1er), (eh12, W1eh)], G, bias_re,
        W2, b2.reshape(1, -1), W3, b3.reshape(1, -1),
        block=2560, bpg=125, want_pmean=True, mean_scale=1.0 / E)
    agg = _sc_scatter_add(e_new, dst3, zeros_n)

    (N1, nb1), (N2, nb2), (N3, nb3) = p['rec_node']
    NWa, NWx, NWt, NWu = N1[0:128], N1[128:384], N1[384:640], N1[640:896]
    bias_rn = _tiny_affine(u_cat, NWu, nb1)

    att1 = _flash_cosine(_pad_rows(x1), _pad_rows(x1h),
                         _pad_rows(x2), _pad_rows(x2h))
    x1n, xm1 = _mlp3_call(
        [(agg[0], NWa), (x1, NWx[:128]), (x1h, NWx[128:]), (att1, NWt)],
        None, bias_rn[0:1], N2, nb2.reshape(1, -1), N3, nb3.reshape(1, -1),
        block=2000, bpg=5, want_pmean=True, mean_scale=1.0 / N)
    u1n = _tiny3(jnp.concatenate([xm1, em[0:1], u_cat[0:1]], 1), *p['rec_glob'])

    att2 = _flash_cosine(_pad_rows(x2), _pad_rows(x2h),
                         _pad_rows(x1), _pad_rows(x1n))
    x2n, xm2 = _mlp3_call(
        [(agg[1], NWa), (x2, NWx[:128]), (x2h, NWx[128:]), (att2, NWt)],
        None, bias_rn[1:2], N2, nb2.reshape(1, -1), N3, nb3.reshape(1, -1),
        block=2000, bpg=5, want_pmean=True, mean_scale=1.0 / N)
    u2n = _tiny3(jnp.concatenate([xm2, em[1:2], u_cat[1:2]], 1), *p['rec_glob'])

    un = jnp.concatenate([u1n, u2n], 0)
    xn = jnp.concatenate([x1n, x2n], 0)

    (A1, ab1), (A2, ab2), (A3, ab3) = p['att_edge']
    A1x, A1e, A1u = A1[0:128], A1[128:256], A1[256:384]
    Pa = _proj_call([(xn, A1x)], block=800)
    Ga = _sc_gather_diff(_pad_table(Pa), dst3g, src3g)
    bias_ae = _tiny_affine(un, A1u, ab1)
    ea, egm, eam = _mlp3_call(
        [(e_new, A1e)], Ga, bias_ae,
        A2, ab2.reshape(1, -1), A3, ab3.reshape(1, -1),
        block=2560, bpg=125, gate=e_new, want_gmean=True, want_pmean=True,
        mean_scale=1.0 / E)
    agga = _sc_scatter_add(ea, dst3, zeros_n)

    (B1, bb1), (B2, bb2), (B3, bb3) = p['att_node']
    B1a, B1x, B1u = B1[0:128], B1[128:256], B1[256:384]
    biasn = _tiny_affine(un, B1u, bb1)
    xgm, xam = _mlp3_call(
        [(agga.reshape(2 * N, 128), B1a), (xn, B1x)], None, biasn,
        B2, bb2.reshape(1, -1), B3, bb3.reshape(1, -1),
        block=2000, bpg=5, gate=xn, want_full=False, want_gmean=True,
        want_pmean=True, mean_scale=1.0 / N)

    ua = _tiny3(jnp.concatenate([xam, eam, un], 1), *p['att_glob'])
    uf = _tiny3(jnp.concatenate([xgm, egm], 1), *p['agg_glob'],
                gate_pair=(un, ua))
    out = _tiny3(uf.reshape(1, 256), *p['final'])
    return out.reshape(64)


# speedup vs baseline: 2.5949x; 1.0221x over previous
import functools

import jax
import jax.numpy as jnp
from jax import lax
from jax.experimental import pallas as pl
from jax.experimental.pallas import tpu as pltpu
from jax.experimental.pallas import tpu_sc as plsc

N = 10000
E = 320000
H = 128
F32 = jnp.float32


def _mlp3_call(segs, extra, bias_pg, W2, b2, W3, b3, *, block, bpg,
               gate=None, want_full=True, want_gmean=False, want_pmean=False,
               mean_scale=1.0):
    R = (extra if extra is not None else segs[0][0]).shape[0]
    grid = R // block
    assert R % block == 0
    ngr = bias_pg.shape[0]
    dout = W3.shape[1]

    inputs = []
    in_specs = []
    for (a, W) in segs:
        d = a.shape[1]
        inputs.append(a)
        in_specs.append(pl.BlockSpec((block, d), lambda i: (i, 0)))
        inputs.append(W)
        in_specs.append(pl.BlockSpec(W.shape, lambda i: (0, 0)))
    if extra is not None:
        inputs.append(extra)
        in_specs.append(pl.BlockSpec((block, 128), lambda i: (i, 0)))
    if gate is not None:
        inputs.append(gate)
        in_specs.append(pl.BlockSpec((block, dout), lambda i: (i, 0)))
    inputs += [bias_pg.reshape(ngr, 1, 128), W2, b2, W3, b3]
    in_specs += [
        pl.BlockSpec((1, 1, 128), lambda i: (i // bpg, 0, 0)),
        pl.BlockSpec(W2.shape, lambda i: (0, 0)),
        pl.BlockSpec((1, 128), lambda i: (0, 0)),
        pl.BlockSpec(W3.shape, lambda i: (0, 0)),
        pl.BlockSpec((1, dout), lambda i: (0, 0)),
    ]

    out_shape = []
    out_specs = []
    if want_full:
        out_shape.append(jax.ShapeDtypeStruct((R, dout), F32))
        out_specs.append(pl.BlockSpec((block, dout), lambda i: (i, 0)))
    if want_gmean:
        out_shape.append(jax.ShapeDtypeStruct((ngr, 1, dout), F32))
        out_specs.append(pl.BlockSpec((1, 1, dout), lambda i: (i // bpg, 0, 0)))
    if want_pmean:
        out_shape.append(jax.ShapeDtypeStruct((ngr, 1, dout), F32))
        out_specs.append(pl.BlockSpec((1, 1, dout), lambda i: (i // bpg, 0, 0)))

    nsegs = len(segs)
    has_extra = extra is not None
    has_gate = gate is not None

    def body(*refs):
        k = 0
        seg_refs = []
        for _ in range(nsegs):
            seg_refs.append((refs[k], refs[k + 1]))
            k += 2
        extra_ref = refs[k] if has_extra else None
        k += 1 if has_extra else 0
        gate_ref = refs[k] if has_gate else None
        k += 1 if has_gate else 0
        b1_ref, W2_ref, b2_ref, W3_ref, b3_ref = refs[k:k + 5]
        k += 5
        o_ref = refs[k] if want_full else None
        k += 1 if want_full else 0
        gm_ref = refs[k] if want_gmean else None
        k += 1 if want_gmean else 0
        pm_ref = refs[k] if want_pmean else None

        i = pl.program_id(0)
        h = jnp.broadcast_to(b1_ref[0], (block, 128))
        for (a_ref, w_ref) in seg_refs:
            h = h + jnp.dot(a_ref[...], w_ref[...], preferred_element_type=F32)
        if has_extra:
            h = h + extra_ref[...]
        h = jnp.maximum(h, 0.0)
        h = jnp.maximum(jnp.dot(h, W2_ref[...], preferred_element_type=F32)
                        + b2_ref[...], 0.0)
        o = jnp.dot(h, W3_ref[...], preferred_element_type=F32) + b3_ref[...]
        if want_full:
            o_ref[...] = o
        if want_gmean:
            v = jnp.sum(o * gate_ref[...], axis=0, keepdims=True)[None]

            @pl.when(i % bpg == 0)
            def _():
                gm_ref[...] = jnp.zeros_like(gm_ref)
            gm_ref[...] += v

            @pl.when(i % bpg == bpg - 1)
            def _():
                gm_ref[...] *= mean_scale
        if want_pmean:
            v2 = jnp.sum(o, axis=0, keepdims=True)[None]

            @pl.when(i % bpg == 0)
            def _():
                pm_ref[...] = jnp.zeros_like(pm_ref)
            pm_ref[...] += v2

            @pl.when(i % bpg == bpg - 1)
            def _():
                pm_ref[...] *= mean_scale

    res = pl.pallas_call(
        body, grid=(grid,), in_specs=in_specs,
        out_specs=out_specs if len(out_specs) > 1 else out_specs[0],
        out_shape=out_shape if len(out_shape) > 1 else out_shape[0],
    )(*inputs)
    res = list(res) if isinstance(res, (tuple, list)) else [res]
    k = 1 if want_full else 0
    for j in range(k, len(res)):
        res[j] = res[j].reshape(ngr, dout)
    return tuple(res)


def _proj_call(segs, *, block):
    R = segs[0][0].shape[0]
    grid = R // block
    inputs = []
    in_specs = []
    for (a, W) in segs:
        d = a.shape[1]
        inputs.append(a)
        in_specs.append(pl.BlockSpec((block, d), lambda i: (i, 0)))
        inputs.append(W)
        in_specs.append(pl.BlockSpec(W.shape, lambda i: (0, 0)))
    nsegs = len(segs)

    def body(*refs):
        acc = None
        for s in range(nsegs):
            a_ref, w_ref = refs[2 * s], refs[2 * s + 1]
            d = jnp.dot(a_ref[...], w_ref[...], preferred_element_type=F32)
            acc = d if acc is None else acc + d
        refs[-1][...] = acc

    return pl.pallas_call(
        body, grid=(grid,), in_specs=in_specs,
        out_specs=pl.BlockSpec((block, 128), lambda i: (i, 0)),
        out_shape=jax.ShapeDtypeStruct((R, 128), F32),
    )(*inputs)


def _tiny3(x, l1, l2, l3, gate_pair=None):
    r = x.shape[0]
    xp = jnp.pad(x, ((0, 8 - r), (0, 0)))
    (W1, b1), (W2, b2), (W3, b3) = l1, l2, l3
    dout = W3.shape[1]
    inputs = [xp, W1, b1.reshape(1, -1), W2, b2.reshape(1, -1),
              W3, b3.reshape(1, -1)]
    if gate_pair is not None:
        ga = jnp.pad(gate_pair[0], ((0, 8 - r), (0, 0)))
        gb = jnp.pad(gate_pair[1], ((0, 8 - r), (0, 0)))
        inputs += [ga, gb]
    has_gate = gate_pair is not None

    def body(*refs):
        x_ref, W1r, b1r, W2r, b2r, W3r, b3r = refs[:7]
        o_ref = refs[-1]
        xv = x_ref[...]
        if has_gate:
            xv = jnp.concatenate([xv, refs[7][...] * refs[8][...]], axis=1)
        h = jnp.maximum(jnp.dot(xv, W1r[...], preferred_element_type=F32)
                        + b1r[...], 0.0)
        h = jnp.maximum(jnp.dot(h, W2r[...], preferred_element_type=F32)
                        + b2r[...], 0.0)
        o_ref[...] = jnp.dot(h, W3r[...], preferred_element_type=F32) + b3r[...]

    out = pl.pallas_call(
        body, out_shape=jax.ShapeDtypeStruct((8, dout), F32),
    )(*inputs)
    return out[:r]


def _tiny_affine(x, W, b):
    r = x.shape[0]
    xp = jnp.pad(x, ((0, 8 - r), (0, 0)))

    def body(x_ref, w_ref, b_ref, o_ref):
        o_ref[...] = (jnp.dot(x_ref[...], w_ref[...], preferred_element_type=F32)
                      + b_ref[...])

    out = pl.pallas_call(
        body, out_shape=jax.ShapeDtypeStruct((8, W.shape[1]), F32),
    )(xp, W, b.reshape(1, -1))
    return out[:r]


_BQ = 512
_BK = 1024
_NPAD = 10240


def _flash_cosine(qr, qh, sr, sh):
    grid_i = _NPAD // _BQ
    grid_j = _NPAD // _BK

    def body(qr_ref, qh_ref, sr_ref, sh_ref, o_ref, acc_ref, m_ref, l_ref):
        j = pl.program_id(1)

        @pl.when(j == 0)
        def _():
            acc_ref[...] = jnp.zeros_like(acc_ref)
            m_ref[...] = jnp.full_like(m_ref, -1e30)
            l_ref[...] = jnp.zeros_like(l_ref)

        q = jnp.concatenate([qr_ref[...], qh_ref[...]], axis=1)
        s = jnp.concatenate([sr_ref[...], sh_ref[...]], axis=1)
        qn = q / (jnp.sqrt(jnp.sum(q * q, axis=1, keepdims=True)) + 1e-8)
        sn = s / (jnp.sqrt(jnp.sum(s * s, axis=1, keepdims=True)) + 1e-8)
        logits = lax.dot_general(qn.astype(jnp.bfloat16), sn.astype(jnp.bfloat16),
                                 (((1,), (1,)), ((), ())),
                                 preferred_element_type=F32)
        col = j * _BK + lax.broadcasted_iota(jnp.int32, (_BQ, _BK), 1)
        logits = jnp.where(col < N, logits, -1e30)
        m_old = m_ref[...]
        m_new = jnp.maximum(m_old, jnp.max(logits, axis=1, keepdims=True))
        alpha = jnp.exp(m_old - m_new)
        p = jnp.exp(logits - m_new)
        l_ref[...] = l_ref[...] * alpha + jnp.sum(p, axis=1, keepdims=True)
        acc_ref[...] = (acc_ref[...] * alpha
                        + jnp.dot(p.astype(jnp.bfloat16), s.astype(jnp.bfloat16),
                                  preferred_element_type=F32))
        m_ref[...] = m_new

        @pl.when(j == grid_j - 1)
        def _():
            o_ref[...] = acc_ref[...] / l_ref[...]

    out = pl.pallas_call(
        body,
        grid=(grid_i, grid_j),
        in_specs=[
            pl.BlockSpec((_BQ, 128), lambda i, j: (i, 0)),
            pl.BlockSpec((_BQ, 128), lambda i, j: (i, 0)),
            pl.BlockSpec((_BK, 128), lambda i, j: (j, 0)),
            pl.BlockSpec((_BK, 128), lambda i, j: (j, 0)),
        ],
        out_specs=pl.BlockSpec((_BQ, 256), lambda i, j: (i, 0)),
        out_shape=jax.ShapeDtypeStruct((_NPAD, 256), F32),
        scratch_shapes=[
            pltpu.VMEM((_BQ, 256), F32),
            pltpu.VMEM((_BQ, 1), F32),
            pltpu.VMEM((_BQ, 1), F32),
        ],
    )(qr, qh, sr, sh)
    return out[:N]


def _pad_rows(a):
    return jnp.pad(a, ((0, _NPAD - a.shape[0]), (0, 0)))


def _sc_gather_diff(table_p, dstl, srcl):
    CH = 80
    n_ch = dstl.shape[1]
    per_t = n_ch * CH
    M = 32 * per_t
    rows_t = _NPAD // 16
    mesh = plsc.VectorSubcoreMesh(core_axis_name="c", subcore_axis_name="s")

    @functools.partial(
        pl.kernel, mesh=mesh,
        out_type=jax.ShapeDtypeStruct((M, 128), F32),
        scratch_types=[
            pltpu.VMEM((CH,), jnp.int32),
            pltpu.VMEM((CH,), jnp.int32),
            pltpu.VMEM((CH,), jnp.int32),
            pltpu.VMEM((CH,), jnp.int32),
            pltpu.VMEM((CH, 128), F32),
            pltpu.VMEM((CH, 128), F32),
            pltpu.VMEM((CH, 128), F32),
            pltpu.VMEM((CH, 128), F32),
            pltpu.SemaphoreType.DMA,
            pltpu.SemaphoreType.DMA,
            pltpu.SemaphoreType.DMA,
            pltpu.SemaphoreType.DMA,
            pltpu.SemaphoreType.DMA,
            pltpu.SemaphoreType.DMA,
            pltpu.SemaphoreType.DMA,
            pltpu.SemaphoreType.DMA,
            pltpu.SemaphoreType.DMA,
            pltpu.SemaphoreType.DMA,
        ])
    def k(table_h, dst_h, src_h, out_h,
          idxd0, idxd1, idxs0, idxs1,
          bufd0, bufd1, bufs0, bufs1,
          gd0, gd1, gs0, gs1, so0, so1, id0, id1, is0, is1):
        c = lax.axis_index("c")
        s = lax.axis_index("s")
        w = c * 16 + s
        base = c * (M // 2) + s * per_t

        idxd = (idxd0, idxd1)
        idxs = (idxs0, idxs1)
        bufd = (bufd0, bufd1)
        bufs = (bufs0, bufs1)
        gsd = (gd0, gd1)
        gss = (gs0, gs1)
        osem = (so0, so1)
        isd = (id0, id1)
        iss = (is0, is1)

        def idx_issue(ci, b):
            pltpu.async_copy(dst_h.at[w, ci], idxd[b], isd[b])
            pltpu.async_copy(src_h.at[w, ci], idxs[b], iss[b])

        def wait_i(b):
            pltpu.make_async_copy(dst_h.at[w, 0], idxd[b], isd[b]).wait()
            pltpu.make_async_copy(src_h.at[w, 0], idxs[b], iss[b]).wait()

        def issue(b):
            pltpu.async_copy(table_h.at[idxd[b]], bufd[b], gsd[b])
            pltpu.async_copy(table_h.at[idxs[b]], bufs[b], gss[b])

        def wait_g(b):
            pltpu.make_async_copy(table_h.at[idxd[b]], bufd[b], gsd[b]).wait()
            pltpu.make_async_copy(table_h.at[idxs[b]], bufs[b], gss[b]).wait()

        def sub(b):
            D, S = bufd[b], bufs[b]

            def row(r, c2):
                for l in range(8):
                    sl = pl.ds(l * 16, 16)
                    D[r, sl] = D[r, sl] - S[r, sl]
                return c2
            lax.fori_loop(0, CH, row, 0, unroll=8)

        def out_issue(ci, b):
            pltpu.async_copy(bufd[b], out_h.at[pl.ds(base + ci * CH, CH)],
                             osem[b])

        def wait_out(b):
            pltpu.make_async_copy(bufd[b], out_h.at[pl.ds(base, CH)],
                                  osem[b]).wait()

        idx_issue(0, 0)
        idx_issue(1, 1)
        wait_i(0)
        issue(0)
        wait_g(0); idx_issue(2, 0); wait_i(1); issue(1); sub(0)
        out_issue(0, 0)
        wait_g(1); idx_issue(3, 1); wait_out(0); wait_i(0); issue(0); sub(1)
        out_issue(1, 1)

        def body(t, carry):
            i0 = 2 * t
            wait_g(0); idx_issue(i0 + 2, 0); wait_out(1); wait_i(1); issue(1)
            sub(0); out_issue(i0, 0)
            wait_g(1); idx_issue(i0 + 3, 1); wait_out(0); wait_i(0); issue(0)
            sub(1); out_issue(i0 + 1, 1)
            return carry
        lax.fori_loop(1, n_ch // 2 - 1, body, 0)

        wait_g(0); wait_out(1); wait_i(1); issue(1); sub(0)
        out_issue(n_ch - 2, 0)
        wait_g(1); wait_out(0); sub(1); out_issue(n_ch - 1, 1)
        wait_out(1)

    return k(table_p, dstl, srcl)


def _sc_scatter_add(vals, dst_local, zeros):
    per_t = E // 16
    CH = 80
    n_ch = per_t // CH
    rows_t = _NPAD // 16
    mesh = plsc.VectorSubcoreMesh(core_axis_name="c", subcore_axis_name="s")

    @functools.partial(
        pl.kernel, mesh=mesh,
        out_type=jax.ShapeDtypeStruct((2, _NPAD, 128), F32),
        scratch_types=[
            pltpu.VMEM((CH,), jnp.int32),
            pltpu.VMEM((CH,), jnp.int32),
            pltpu.VMEM((CH, 128), F32),
            pltpu.VMEM((CH, 128), F32),
            pltpu.VMEM_SHARED((_NPAD, 128), F32),
            pltpu.SemaphoreType.DMA,
            pltpu.SemaphoreType.DMA,
            pltpu.SemaphoreType.DMA,
            pltpu.SemaphoreType.DMA,
        ])
    def k(vals_h, dst_h, zeros_h, out_h, idx0, idx1, vbuf0, vbuf1, acc,
          vs0, vs1, is0, is1):
        c = lax.axis_index("c")
        s = lax.axis_index("s")
        w = c * 16 + s
        for j in range(rows_t // CH):
            sl = pl.ds(s * rows_t + j * CH, CH)
            pltpu.sync_copy(zeros_h.at[sl], vbuf0)
            pltpu.sync_copy(vbuf0, acc.at[sl])
        plsc.subcore_barrier()
        base = c * E + s * per_t

        idxb = (idx0, idx1)
        isem = (is0, is1)
        vbuf = (vbuf0, vbuf1)
        vsem = (vs0, vs1)

        def idx_issue(ci, b):
            pltpu.async_copy(dst_h.at[w, ci], idxb[b], isem[b])

        def wait_i(b):
            pltpu.make_async_copy(dst_h.at[w, 0], idxb[b], isem[b]).wait()

        def load_issue(ci, b):
            pltpu.async_copy(vals_h.at[pl.ds(base + ci * CH, CH)], vbuf[b],
                             vsem[b])

        def wait_v(b):
            pltpu.make_async_copy(vals_h.at[pl.ds(base, CH)], vbuf[b],
                                  vsem[b]).wait()

        def scat(b):
            pltpu.sync_copy(vbuf[b], acc.at[idxb[b]], add=True)

        idx_issue(0, 0)
        idx_issue(1, 1)
        load_issue(0, 0)

        def body(t, carry):
            i0 = 2 * t
            load_issue(i0 + 1, 1); wait_v(0); wait_i(0); scat(0)
            idx_issue(i0 + 2, 0)
            load_issue(i0 + 2, 0); wait_v(1); wait_i(1); scat(1)
            idx_issue(i0 + 3, 1)
            return carry
        lax.fori_loop(0, n_ch // 2 - 1, body, 0)

        load_issue(n_ch - 1, 1); wait_v(0); wait_i(0); scat(0)
        wait_v(1); wait_i(1); scat(1)

        plsc.subcore_barrier()
        for j in range(rows_t // CH):
            sl = pl.ds(s * rows_t + j * CH, CH)
            pltpu.sync_copy(acc.at[sl], vbuf0)
            pltpu.sync_copy(vbuf0, out_h.at[c, sl])

    return k(vals, dst_local, zeros)[:, :N]


def kernel(x1, edge_index1, e1, u1, x2, edge_index2, e2, u2, params):
    p = params

    x12 = jnp.concatenate([x1, x2], 0)
    e12 = jnp.concatenate([e1, e2], 0)
    src_l = jnp.concatenate([edge_index1[0], edge_index2[0]])
    dst_l = jnp.concatenate([edge_index1[1], edge_index2[1]])
    src3 = src_l.reshape(32, 250, 80)
    dst3 = dst_l.reshape(32, 250, 80)
    src3g = jnp.concatenate(
        [edge_index1[0], edge_index2[0] + _NPAD]).reshape(32, 250, 80)
    dst3g = jnp.concatenate(
        [edge_index1[1], edge_index2[1] + _NPAD]).reshape(32, 250, 80)
    zeros_n = jnp.zeros((_NPAD, 128), F32)
    zpad = jnp.zeros((_NPAD - N, 128), F32)

    def _pad_table(t):
        return jnp.concatenate([t[:N], zpad, t[N:], zpad], 0)

    (enW1, enb1), (enW2, enb2), (enW3, enb3) = p['enc_node']
    (xh12,) = _mlp3_call([(x12, enW1)], None, enb1.reshape(1, -1),
                         enW2, enb2.reshape(1, -1), enW3, enb3.reshape(1, -1),
                         block=800, bpg=25)
    (eeW1, eeb1), (eeW2, eeb2), (eeW3, eeb3) = p['enc_edge']
    (eh12,) = _mlp3_call([(e12, eeW1)], None, eeb1.reshape(1, -1),
                         eeW2, eeb2.reshape(1, -1), eeW3, eeb3.reshape(1, -1),
                         block=2560, bpg=250)
    uh = _tiny3(jnp.stack([u1, u2]), *p['enc_glob'])
    u_cat = jnp.concatenate([jnp.stack([u1, u2]), uh], 1)

    x1h, x2h = xh12[:N], xh12[N:]

    (W1, b1), (W2, b2), (W3, b3) = p['rec_edge']
    W1x, W1er, W1eh, W1u = W1[0:256], W1[256:272], W1[272:400], W1[400:656]
    Pt = _proj_call([(x12, W1x[:128]), (xh12, W1x[128:])], block=800)
    G = _sc_gather_diff(_pad_table(Pt), dst3g, src3g)
    bias_re = _tiny_affine(u_cat, W1u, b1)
    e_new, em = _mlp3_call(
        [(e12, W1er), (eh12, W1eh)], G, bias_re,
        W2, b2.reshape(1, -1), W3, b3.reshape(1, -1),
        block=2560, bpg=125, want_pmean=True, mean_scale=1.0 / E)
    agg = _sc_scatter_add(e_new, dst3, zeros_n)

    (N1, nb1), (N2, nb2), (N3, nb3) = p['rec_node']
    NWa, NWx, NWt, NWu = N1[0:128], N1[128:384], N1[384:640], N1[640:896]
    bias_rn = _tiny_affine(u_cat, NWu, nb1)

    att1 = _flash_cosine(_pad_rows(x1), _pad_rows(x1h),
                         _pad_rows(x2), _pad_rows(x2h))
    x1n, xm1 = _mlp3_call(
        [(agg[0], NWa), (x1, NWx[:128]), (x1h, NWx[128:]), (att1, NWt)],
        None, bias_rn[0:1], N2, nb2.reshape(1, -1), N3, nb3.reshape(1, -1),
        block=2000, bpg=5, want_pmean=True, mean_scale=1.0 / N)
    u1n = _tiny3(jnp.concatenate([xm1, em[0:1], u_cat[0:1]], 1), *p['rec_glob'])

    att2 = _flash_cosine(_pad_rows(x2), _pad_rows(x2h),
                         _pad_rows(x1), _pad_rows(x1n))
    x2n, xm2 = _mlp3_call(
        [(agg[1], NWa), (x2, NWx[:128]), (x2h, NWx[128:]), (att2, NWt)],
        None, bias_rn[1:2], N2, nb2.reshape(1, -1), N3, nb3.reshape(1, -1),
        block=2000, bpg=5, want_pmean=True, mean_scale=1.0 / N)
    u2n = _tiny3(jnp.concatenate([xm2, em[1:2], u_cat[1:2]], 1), *p['rec_glob'])

    un = jnp.concatenate([u1n, u2n], 0)
    xn = jnp.concatenate([x1n, x2n], 0)

    (A1, ab1), (A2, ab2), (A3, ab3) = p['att_edge']
    A1x, A1e, A1u = A1[0:128], A1[128:256], A1[256:384]
    Pa = _proj_call([(xn, A1x)], block=800)
    Ga = _sc_gather_diff(_pad_table(Pa), dst3g, src3g)
    bias_ae = _tiny_affine(un, A1u, ab1)
    ea, egm, eam = _mlp3_call(
        [(e_new, A1e)], Ga, bias_ae,
        A2, ab2.reshape(1, -1), A3, ab3.reshape(1, -1),
        block=2560, bpg=125, gate=e_new, want_gmean=True, want_pmean=True,
        mean_scale=1.0 / E)
    agga = _sc_scatter_add(ea, dst3, zeros_n)

    (B1, bb1), (B2, bb2), (B3, bb3) = p['att_node']
    B1a, B1x, B1u = B1[0:128], B1[128:256], B1[256:384]
    biasn = _tiny_affine(un, B1u, bb1)
    xgm, xam = _mlp3_call(
        [(agga.reshape(2 * N, 128), B1a), (xn, B1x)], None, biasn,
        B2, bb2.reshape(1, -1), B3, bb3.reshape(1, -1),
        block=2000, bpg=5, gate=xn, want_full=False, want_gmean=True,
        want_pmean=True, mean_scale=1.0 / N)

    ua = _tiny3(jnp.concatenate([xam, eam, un], 1), *p['att_glob'])
    uf = _tiny3(jnp.concatenate([xgm, egm], 1), *p['agg_glob'],
                gate_pair=(un, ua))
    out = _tiny3(uf.reshape(1, 256), *p['final'])
    return out.reshape(64)


# speedup vs baseline: 2.7282x; 1.0514x over previous
import functools

import jax
import jax.numpy as jnp
from jax import lax
from jax.experimental import pallas as pl
from jax.experimental.pallas import tpu as pltpu
from jax.experimental.pallas import tpu_sc as plsc

N = 10000
E = 320000
H = 128
F32 = jnp.float32


def _mlp3_call(segs, extra, bias_pg, W2, b2, W3, b3, *, block, bpg,
               gate=None, want_full=True, want_gmean=False, want_pmean=False,
               mean_scale=1.0):
    R = (extra if extra is not None else segs[0][0]).shape[0]
    grid = R // block
    assert R % block == 0
    ngr = bias_pg.shape[0]
    dout = W3.shape[1]

    inputs = []
    in_specs = []
    for (a, W) in segs:
        d = a.shape[1]
        inputs.append(a)
        in_specs.append(pl.BlockSpec((block, d), lambda i: (i, 0)))
        inputs.append(W)
        in_specs.append(pl.BlockSpec(W.shape, lambda i: (0, 0)))
    if extra is not None:
        inputs.append(extra)
        in_specs.append(pl.BlockSpec((block, 128), lambda i: (i, 0)))
    if gate is not None:
        inputs.append(gate)
        in_specs.append(pl.BlockSpec((block, dout), lambda i: (i, 0)))
    inputs += [bias_pg.reshape(ngr, 1, 128), W2, b2, W3, b3]
    in_specs += [
        pl.BlockSpec((1, 1, 128), lambda i: (i // bpg, 0, 0)),
        pl.BlockSpec(W2.shape, lambda i: (0, 0)),
        pl.BlockSpec((1, 128), lambda i: (0, 0)),
        pl.BlockSpec(W3.shape, lambda i: (0, 0)),
        pl.BlockSpec((1, dout), lambda i: (0, 0)),
    ]

    out_shape = []
    out_specs = []
    if want_full:
        out_shape.append(jax.ShapeDtypeStruct((R, dout), F32))
        out_specs.append(pl.BlockSpec((block, dout), lambda i: (i, 0)))
    if want_gmean:
        out_shape.append(jax.ShapeDtypeStruct((ngr, 1, dout), F32))
        out_specs.append(pl.BlockSpec((1, 1, dout), lambda i: (i // bpg, 0, 0)))
    if want_pmean:
        out_shape.append(jax.ShapeDtypeStruct((ngr, 1, dout), F32))
        out_specs.append(pl.BlockSpec((1, 1, dout), lambda i: (i // bpg, 0, 0)))

    nsegs = len(segs)
    has_extra = extra is not None
    has_gate = gate is not None

    def body(*refs):
        k = 0
        seg_refs = []
        for _ in range(nsegs):
            seg_refs.append((refs[k], refs[k + 1]))
            k += 2
        extra_ref = refs[k] if has_extra else None
        k += 1 if has_extra else 0
        gate_ref = refs[k] if has_gate else None
        k += 1 if has_gate else 0
        b1_ref, W2_ref, b2_ref, W3_ref, b3_ref = refs[k:k + 5]
        k += 5
        o_ref = refs[k] if want_full else None
        k += 1 if want_full else 0
        gm_ref = refs[k] if want_gmean else None
        k += 1 if want_gmean else 0
        pm_ref = refs[k] if want_pmean else None

        i = pl.program_id(0)
        h = jnp.broadcast_to(b1_ref[0], (block, 128))
        for (a_ref, w_ref) in seg_refs:
            h = h + jnp.dot(a_ref[...], w_ref[...], preferred_element_type=F32)
        if has_extra:
            h = h + extra_ref[...]
        h = jnp.maximum(h, 0.0)
        h = jnp.maximum(jnp.dot(h, W2_ref[...], preferred_element_type=F32)
                        + b2_ref[...], 0.0)
        o = jnp.dot(h, W3_ref[...], preferred_element_type=F32) + b3_ref[...]
        if want_full:
            o_ref[...] = o
        if want_gmean:
            v = jnp.sum(o * gate_ref[...], axis=0, keepdims=True)[None]

            @pl.when(i % bpg == 0)
            def _():
                gm_ref[...] = jnp.zeros_like(gm_ref)
            gm_ref[...] += v

            @pl.when(i % bpg == bpg - 1)
            def _():
                gm_ref[...] *= mean_scale
        if want_pmean:
            v2 = jnp.sum(o, axis=0, keepdims=True)[None]

            @pl.when(i % bpg == 0)
            def _():
                pm_ref[...] = jnp.zeros_like(pm_ref)
            pm_ref[...] += v2

            @pl.when(i % bpg == bpg - 1)
            def _():
                pm_ref[...] *= mean_scale

    res = pl.pallas_call(
        body, grid=(grid,), in_specs=in_specs,
        out_specs=out_specs if len(out_specs) > 1 else out_specs[0],
        out_shape=out_shape if len(out_shape) > 1 else out_shape[0],
    )(*inputs)
    res = list(res) if isinstance(res, (tuple, list)) else [res]
    k = 1 if want_full else 0
    for j in range(k, len(res)):
        res[j] = res[j].reshape(ngr, dout)
    return tuple(res)


def _proj_call(segs, *, block):
    R = segs[0][0].shape[0]
    grid = R // block
    inputs = []
    in_specs = []
    for (a, W) in segs:
        d = a.shape[1]
        inputs.append(a)
        in_specs.append(pl.BlockSpec((block, d), lambda i: (i, 0)))
        inputs.append(W)
        in_specs.append(pl.BlockSpec(W.shape, lambda i: (0, 0)))
    nsegs = len(segs)

    def body(*refs):
        acc = None
        for s in range(nsegs):
            a_ref, w_ref = refs[2 * s], refs[2 * s + 1]
            d = jnp.dot(a_ref[...], w_ref[...], preferred_element_type=F32)
            acc = d if acc is None else acc + d
        refs[-1][...] = acc

    return pl.pallas_call(
        body, grid=(grid,), in_specs=in_specs,
        out_specs=pl.BlockSpec((block, 128), lambda i: (i, 0)),
        out_shape=jax.ShapeDtypeStruct((R, 128), F32),
    )(*inputs)


def _tiny3(x, l1, l2, l3, gate_pair=None):
    r = x.shape[0]
    xp = jnp.pad(x, ((0, 8 - r), (0, 0)))
    (W1, b1), (W2, b2), (W3, b3) = l1, l2, l3
    dout = W3.shape[1]
    inputs = [xp, W1, b1.reshape(1, -1), W2, b2.reshape(1, -1),
              W3, b3.reshape(1, -1)]
    if gate_pair is not None:
        ga = jnp.pad(gate_pair[0], ((0, 8 - r), (0, 0)))
        gb = jnp.pad(gate_pair[1], ((0, 8 - r), (0, 0)))
        inputs += [ga, gb]
    has_gate = gate_pair is not None

    def body(*refs):
        x_ref, W1r, b1r, W2r, b2r, W3r, b3r = refs[:7]
        o_ref = refs[-1]
        xv = x_ref[...]
        if has_gate:
            xv = jnp.concatenate([xv, refs[7][...] * refs[8][...]], axis=1)
        h = jnp.maximum(jnp.dot(xv, W1r[...], preferred_element_type=F32)
                        + b1r[...], 0.0)
        h = jnp.maximum(jnp.dot(h, W2r[...], preferred_element_type=F32)
                        + b2r[...], 0.0)
        o_ref[...] = jnp.dot(h, W3r[...], preferred_element_type=F32) + b3r[...]

    out = pl.pallas_call(
        body, out_shape=jax.ShapeDtypeStruct((8, dout), F32),
    )(*inputs)
    return out[:r]


def _tiny_affine(x, W, b):
    r = x.shape[0]
    xp = jnp.pad(x, ((0, 8 - r), (0, 0)))

    def body(x_ref, w_ref, b_ref, o_ref):
        o_ref[...] = (jnp.dot(x_ref[...], w_ref[...], preferred_element_type=F32)
                      + b_ref[...])

    out = pl.pallas_call(
        body, out_shape=jax.ShapeDtypeStruct((8, W.shape[1]), F32),
    )(xp, W, b.reshape(1, -1))
    return out[:r]


_BQ = 1024
_BK = 1024
_NPAD = 10240


def _flash_cosine(qr, qh, sr, sh):
    grid_i = _NPAD // _BQ
    grid_j = _NPAD // _BK

    def body(qr_ref, qh_ref, sr_ref, sh_ref, o_ref, acc_ref, m_ref, l_ref):
        j = pl.program_id(1)

        @pl.when(j == 0)
        def _():
            acc_ref[...] = jnp.zeros_like(acc_ref)
            m_ref[...] = jnp.full_like(m_ref, -1e30)
            l_ref[...] = jnp.zeros_like(l_ref)

        q = jnp.concatenate([qr_ref[...], qh_ref[...]], axis=1)
        s = jnp.concatenate([sr_ref[...], sh_ref[...]], axis=1)
        qn = q / (jnp.sqrt(jnp.sum(q * q, axis=1, keepdims=True)) + 1e-8)
        sn = s / (jnp.sqrt(jnp.sum(s * s, axis=1, keepdims=True)) + 1e-8)
        logits = lax.dot_general(qn.astype(jnp.bfloat16), sn.astype(jnp.bfloat16),
                                 (((1,), (1,)), ((), ())),
                                 preferred_element_type=F32)
        col = j * _BK + lax.broadcasted_iota(jnp.int32, (_BQ, _BK), 1)
        logits = jnp.where(col < N, logits, -1e30)
        m_old = m_ref[...]
        m_new = jnp.maximum(m_old, jnp.max(logits, axis=1, keepdims=True))
        alpha = jnp.exp(m_old - m_new)
        p = jnp.exp(logits - m_new)
        l_ref[...] = l_ref[...] * alpha + jnp.sum(p, axis=1, keepdims=True)
        acc_ref[...] = (acc_ref[...] * alpha
                        + jnp.dot(p.astype(jnp.bfloat16), s.astype(jnp.bfloat16),
                                  preferred_element_type=F32))
        m_ref[...] = m_new

        @pl.when(j == grid_j - 1)
        def _():
            o_ref[...] = acc_ref[...] / l_ref[...]

    out = pl.pallas_call(
        body,
        grid=(grid_i, grid_j),
        in_specs=[
            pl.BlockSpec((_BQ, 128), lambda i, j: (i, 0)),
            pl.BlockSpec((_BQ, 128), lambda i, j: (i, 0)),
            pl.BlockSpec((_BK, 128), lambda i, j: (j, 0)),
            pl.BlockSpec((_BK, 128), lambda i, j: (j, 0)),
        ],
        out_specs=pl.BlockSpec((_BQ, 256), lambda i, j: (i, 0)),
        out_shape=jax.ShapeDtypeStruct((_NPAD, 256), F32),
        scratch_shapes=[
            pltpu.VMEM((_BQ, 256), F32),
            pltpu.VMEM((_BQ, 1), F32),
            pltpu.VMEM((_BQ, 1), F32),
        ],
    )(qr, qh, sr, sh)
    return out[:N]


def _pad_rows(a):
    return jnp.pad(a, ((0, _NPAD - a.shape[0]), (0, 0)))


def _sc_gather_diff(table_p, dstl, srcl):
    CH = 80
    n_ch = dstl.shape[1]
    per_t = n_ch * CH
    M = 32 * per_t
    rows_t = _NPAD // 16
    mesh = plsc.VectorSubcoreMesh(core_axis_name="c", subcore_axis_name="s")

    @functools.partial(
        pl.kernel, mesh=mesh,
        out_type=jax.ShapeDtypeStruct((M, 128), F32),
        scratch_types=[
            pltpu.VMEM((CH,), jnp.int32),
            pltpu.VMEM((CH,), jnp.int32),
            pltpu.VMEM((CH,), jnp.int32),
            pltpu.VMEM((CH,), jnp.int32),
            pltpu.VMEM((CH, 128), F32),
            pltpu.VMEM((CH, 128), F32),
            pltpu.VMEM((CH, 128), F32),
            pltpu.VMEM((CH, 128), F32),
            pltpu.SemaphoreType.DMA,
            pltpu.SemaphoreType.DMA,
            pltpu.SemaphoreType.DMA,
            pltpu.SemaphoreType.DMA,
            pltpu.SemaphoreType.DMA,
            pltpu.SemaphoreType.DMA,
            pltpu.SemaphoreType.DMA,
            pltpu.SemaphoreType.DMA,
            pltpu.SemaphoreType.DMA,
            pltpu.SemaphoreType.DMA,
        ])
    def k(table_h, dst_h, src_h, out_h,
          idxd0, idxd1, idxs0, idxs1,
          bufd0, bufd1, bufs0, bufs1,
          gd0, gd1, gs0, gs1, so0, so1, id0, id1, is0, is1):
        c = lax.axis_index("c")
        s = lax.axis_index("s")
        w = c * 16 + s
        base = c * (M // 2) + s * per_t

        idxd = (idxd0, idxd1)
        idxs = (idxs0, idxs1)
        bufd = (bufd0, bufd1)
        bufs = (bufs0, bufs1)
        gsd = (gd0, gd1)
        gss = (gs0, gs1)
        osem = (so0, so1)
        isd = (id0, id1)
        iss = (is0, is1)

        def idx_issue(ci, b):
            pltpu.async_copy(dst_h.at[w, ci], idxd[b], isd[b])
            pltpu.async_copy(src_h.at[w, ci], idxs[b], iss[b])

        def wait_i(b):
            pltpu.make_async_copy(dst_h.at[w, 0], idxd[b], isd[b]).wait()
            pltpu.make_async_copy(src_h.at[w, 0], idxs[b], iss[b]).wait()

        def issue(b):
            pltpu.async_copy(table_h.at[idxd[b]], bufd[b], gsd[b])
            pltpu.async_copy(table_h.at[idxs[b]], bufs[b], gss[b])

        def wait_g(b):
            pltpu.make_async_copy(table_h.at[idxd[b]], bufd[b], gsd[b]).wait()
            pltpu.make_async_copy(table_h.at[idxs[b]], bufs[b], gss[b]).wait()

        def sub(b):
            D, S = bufd[b], bufs[b]

            def row(r, c2):
                for l in range(8):
                    sl = pl.ds(l * 16, 16)
                    D[r, sl] = D[r, sl] - S[r, sl]
                return c2
            lax.fori_loop(0, CH, row, 0, unroll=8)

        def out_issue(ci, b):
            pltpu.async_copy(bufd[b], out_h.at[pl.ds(base + ci * CH, CH)],
                             osem[b])

        def wait_out(b):
            pltpu.make_async_copy(bufd[b], out_h.at[pl.ds(base, CH)],
                                  osem[b]).wait()

        idx_issue(0, 0)
        idx_issue(1, 1)
        wait_i(0)
        issue(0)
        wait_g(0); idx_issue(2, 0); wait_i(1); issue(1); sub(0)
        out_issue(0, 0)
        wait_g(1); idx_issue(3, 1); wait_out(0); wait_i(0); issue(0); sub(1)
        out_issue(1, 1)

        def body(t, carry):
            i0 = 2 * t
            wait_g(0); idx_issue(i0 + 2, 0); wait_out(1); wait_i(1); issue(1)
            sub(0); out_issue(i0, 0)
            wait_g(1); idx_issue(i0 + 3, 1); wait_out(0); wait_i(0); issue(0)
            sub(1); out_issue(i0 + 1, 1)
            return carry
        lax.fori_loop(1, n_ch // 2 - 1, body, 0)

        wait_g(0); wait_out(1); wait_i(1); issue(1); sub(0)
        out_issue(n_ch - 2, 0)
        wait_g(1); wait_out(0); sub(1); out_issue(n_ch - 1, 1)
        wait_out(1)

    return k(table_p, dstl, srcl)


def _sc_scatter_add(vals, dst_local, zeros):
    per_t = E // 16
    CH = 80
    n_ch = per_t // CH
    rows_t = _NPAD // 16
    mesh = plsc.VectorSubcoreMesh(core_axis_name="c", subcore_axis_name="s")

    @functools.partial(
        pl.kernel, mesh=mesh,
        out_type=jax.ShapeDtypeStruct((2, _NPAD, 128), F32),
        scratch_types=[
            pltpu.VMEM((CH,), jnp.int32),
            pltpu.VMEM((CH,), jnp.int32),
            pltpu.VMEM((CH, 128), F32),
            pltpu.VMEM((CH, 128), F32),
            pltpu.VMEM_SHARED((_NPAD, 128), F32),
            pltpu.SemaphoreType.DMA,
            pltpu.SemaphoreType.DMA,
            pltpu.SemaphoreType.DMA,
            pltpu.SemaphoreType.DMA,
        ])
    def k(vals_h, dst_h, zeros_h, out_h, idx0, idx1, vbuf0, vbuf1, acc,
          vs0, vs1, is0, is1):
        c = lax.axis_index("c")
        s = lax.axis_index("s")
        w = c * 16 + s
        for j in range(rows_t // CH):
            sl = pl.ds(s * rows_t + j * CH, CH)
            pltpu.sync_copy(zeros_h.at[sl], vbuf0)
            pltpu.sync_copy(vbuf0, acc.at[sl])
        plsc.subcore_barrier()
        base = c * E + s * per_t

        idxb = (idx0, idx1)
        isem = (is0, is1)
        vbuf = (vbuf0, vbuf1)
        vsem = (vs0, vs1)

        def idx_issue(ci, b):
            pltpu.async_copy(dst_h.at[w, ci], idxb[b], isem[b])

        def wait_i(b):
            pltpu.make_async_copy(dst_h.at[w, 0], idxb[b], isem[b]).wait()

        def load_issue(ci, b):
            pltpu.async_copy(vals_h.at[pl.ds(base + ci * CH, CH)], vbuf[b],
                             vsem[b])

        def wait_v(b):
            pltpu.make_async_copy(vals_h.at[pl.ds(base, CH)], vbuf[b],
                                  vsem[b]).wait()

        def scat(b):
            pltpu.sync_copy(vbuf[b], acc.at[idxb[b]], add=True)

        idx_issue(0, 0)
        idx_issue(1, 1)
        load_issue(0, 0)

        def body(t, carry):
            i0 = 2 * t
            load_issue(i0 + 1, 1); wait_v(0); wait_i(0); scat(0)
            idx_issue(i0 + 2, 0)
            load_issue(i0 + 2, 0); wait_v(1); wait_i(1); scat(1)
            idx_issue(i0 + 3, 1)
            return carry
        lax.fori_loop(0, n_ch // 2 - 1, body, 0)

        load_issue(n_ch - 1, 1); wait_v(0); wait_i(0); scat(0)
        wait_v(1); wait_i(1); scat(1)

        plsc.subcore_barrier()
        for j in range(rows_t // CH):
            sl = pl.ds(s * rows_t + j * CH, CH)
            pltpu.sync_copy(acc.at[sl], vbuf0)
            pltpu.sync_copy(vbuf0, out_h.at[c, sl])

    return k(vals, dst_local, zeros)[:, :N]


def kernel(x1, edge_index1, e1, u1, x2, edge_index2, e2, u2, params):
    p = params

    x12 = jnp.concatenate([x1, x2], 0)
    e12 = jnp.concatenate([e1, e2], 0)
    src_l = jnp.concatenate([edge_index1[0], edge_index2[0]])
    dst_l = jnp.concatenate([edge_index1[1], edge_index2[1]])
    src3 = src_l.reshape(32, 250, 80)
    dst3 = dst_l.reshape(32, 250, 80)
    src3g = jnp.concatenate(
        [edge_index1[0], edge_index2[0] + _NPAD]).reshape(32, 250, 80)
    dst3g = jnp.concatenate(
        [edge_index1[1], edge_index2[1] + _NPAD]).reshape(32, 250, 80)
    zeros_n = jnp.zeros((_NPAD, 128), F32)
    zpad = jnp.zeros((_NPAD - N, 128), F32)

    def _pad_table(t):
        return jnp.concatenate([t[:N], zpad, t[N:], zpad], 0)

    (enW1, enb1), (enW2, enb2), (enW3, enb3) = p['enc_node']
    (xh12,) = _mlp3_call([(x12, enW1)], None, enb1.reshape(1, -1),
                         enW2, enb2.reshape(1, -1), enW3, enb3.reshape(1, -1),
                         block=800, bpg=25)
    (eeW1, eeb1), (eeW2, eeb2), (eeW3, eeb3) = p['enc_edge']
    (eh12,) = _mlp3_call([(e12, eeW1)], None, eeb1.reshape(1, -1),
                         eeW2, eeb2.reshape(1, -1), eeW3, eeb3.reshape(1, -1),
                         block=4000, bpg=160)
    uh = _tiny3(jnp.stack([u1, u2]), *p['enc_glob'])
    u_cat = jnp.concatenate([jnp.stack([u1, u2]), uh], 1)

    x1h, x2h = xh12[:N], xh12[N:]

    (W1, b1), (W2, b2), (W3, b3) = p['rec_edge']
    W1x, W1er, W1eh, W1u = W1[0:256], W1[256:272], W1[272:400], W1[400:656]
    Pt = _proj_call([(x12, W1x[:128]), (xh12, W1x[128:])], block=800)
    G = _sc_gather_diff(_pad_table(Pt), dst3g, src3g)
    bias_re = _tiny_affine(u_cat, W1u, b1)
    e_new, em = _mlp3_call(
        [(e12, W1er), (eh12, W1eh)], G, bias_re,
        W2, b2.reshape(1, -1), W3, b3.reshape(1, -1),
        block=4000, bpg=80, want_pmean=True, mean_scale=1.0 / E)
    agg = _sc_scatter_add(e_new, dst3, zeros_n)

    (N1, nb1), (N2, nb2), (N3, nb3) = p['rec_node']
    NWa, NWx, NWt, NWu = N1[0:128], N1[128:384], N1[384:640], N1[640:896]
    bias_rn = _tiny_affine(u_cat, NWu, nb1)

    att1 = _flash_cosine(_pad_rows(x1), _pad_rows(x1h),
                         _pad_rows(x2), _pad_rows(x2h))
    x1n, xm1 = _mlp3_call(
        [(agg[0], NWa), (x1, NWx[:128]), (x1h, NWx[128:]), (att1, NWt)],
        None, bias_rn[0:1], N2, nb2.reshape(1, -1), N3, nb3.reshape(1, -1),
        block=2000, bpg=5, want_pmean=True, mean_scale=1.0 / N)
    u1n = _tiny3(jnp.concatenate([xm1, em[0:1], u_cat[0:1]], 1), *p['rec_glob'])

    att2 = _flash_cosine(_pad_rows(x2), _pad_rows(x2h),
                         _pad_rows(x1), _pad_rows(x1n))
    x2n, xm2 = _mlp3_call(
        [(agg[1], NWa), (x2, NWx[:128]), (x2h, NWx[128:]), (att2, NWt)],
        None, bias_rn[1:2], N2, nb2.reshape(1, -1), N3, nb3.reshape(1, -1),
        block=2000, bpg=5, want_pmean=True, mean_scale=1.0 / N)
    u2n = _tiny3(jnp.concatenate([xm2, em[1:2], u_cat[1:2]], 1), *p['rec_glob'])

    un = jnp.concatenate([u1n, u2n], 0)
    xn = jnp.concatenate([x1n, x2n], 0)

    (A1, ab1), (A2, ab2), (A3, ab3) = p['att_edge']
    A1x, A1e, A1u = A1[0:128], A1[128:256], A1[256:384]
    Pa = _proj_call([(xn, A1x)], block=800)
    Ga = _sc_gather_diff(_pad_table(Pa), dst3g, src3g)
    bias_ae = _tiny_affine(un, A1u, ab1)
    ea, egm, eam = _mlp3_call(
        [(e_new, A1e)], Ga, bias_ae,
        A2, ab2.reshape(1, -1), A3, ab3.reshape(1, -1),
        block=4000, bpg=80, gate=e_new, want_gmean=True, want_pmean=True,
        mean_scale=1.0 / E)
    agga = _sc_scatter_add(ea, dst3, zeros_n)

    (B1, bb1), (B2, bb2), (B3, bb3) = p['att_node']
    B1a, B1x, B1u = B1[0:128], B1[128:256], B1[256:384]
    biasn = _tiny_affine(un, B1u, bb1)
    xgm, xam = _mlp3_call(
        [(agga.reshape(2 * N, 128), B1a), (xn, B1x)], None, biasn,
        B2, bb2.reshape(1, -1), B3, bb3.reshape(1, -1),
        block=2000, bpg=5, gate=xn, want_full=False, want_gmean=True,
        want_pmean=True, mean_scale=1.0 / N)

    ua = _tiny3(jnp.concatenate([xam, eam, un], 1), *p['att_glob'])
    uf = _tiny3(jnp.concatenate([xgm, egm], 1), *p['agg_glob'],
                gate_pair=(un, ua))
    out = _tiny3(uf.reshape(1, 256), *p['final'])
    return out.reshape(64)


# speedup vs baseline: 2.8480x; 1.0439x over previous
import functools

import jax
import jax.numpy as jnp
from jax import lax
from jax.experimental import pallas as pl
from jax.experimental.pallas import tpu as pltpu
from jax.experimental.pallas import tpu_sc as plsc

N = 10000
E = 320000
H = 128
F32 = jnp.float32


def _mlp3_call(segs, extra, bias_pg, W2, b2, W3, b3, *, block, bpg,
               gate=None, want_full=True, want_gmean=False, want_pmean=False,
               mean_scale=1.0):
    R = (extra if extra is not None else segs[0][0]).shape[0]
    grid = R // block
    assert R % block == 0
    ngr = bias_pg.shape[0]
    dout = W3.shape[1]

    inputs = []
    in_specs = []
    for (a, W) in segs:
        d = a.shape[1]
        inputs.append(a)
        in_specs.append(pl.BlockSpec((block, d), lambda i: (i, 0)))
        inputs.append(W)
        in_specs.append(pl.BlockSpec(W.shape, lambda i: (0, 0)))
    if extra is not None:
        inputs.append(extra)
        in_specs.append(pl.BlockSpec((block, 128), lambda i: (i, 0)))
    if gate is not None:
        inputs.append(gate)
        in_specs.append(pl.BlockSpec((block, dout), lambda i: (i, 0)))
    inputs += [bias_pg.reshape(ngr, 1, 128), W2, b2, W3, b3]
    in_specs += [
        pl.BlockSpec((1, 1, 128), lambda i: (i // bpg, 0, 0)),
        pl.BlockSpec(W2.shape, lambda i: (0, 0)),
        pl.BlockSpec((1, 128), lambda i: (0, 0)),
        pl.BlockSpec(W3.shape, lambda i: (0, 0)),
        pl.BlockSpec((1, dout), lambda i: (0, 0)),
    ]

    out_shape = []
    out_specs = []
    if want_full:
        out_shape.append(jax.ShapeDtypeStruct((R, dout), F32))
        out_specs.append(pl.BlockSpec((block, dout), lambda i: (i, 0)))
    if want_gmean:
        out_shape.append(jax.ShapeDtypeStruct((ngr, 1, dout), F32))
        out_specs.append(pl.BlockSpec((1, 1, dout), lambda i: (i // bpg, 0, 0)))
    if want_pmean:
        out_shape.append(jax.ShapeDtypeStruct((ngr, 1, dout), F32))
        out_specs.append(pl.BlockSpec((1, 1, dout), lambda i: (i // bpg, 0, 0)))

    nsegs = len(segs)
    has_extra = extra is not None
    has_gate = gate is not None

    def body(*refs):
        k = 0
        seg_refs = []
        for _ in range(nsegs):
            seg_refs.append((refs[k], refs[k + 1]))
            k += 2
        extra_ref = refs[k] if has_extra else None
        k += 1 if has_extra else 0
        gate_ref = refs[k] if has_gate else None
        k += 1 if has_gate else 0
        b1_ref, W2_ref, b2_ref, W3_ref, b3_ref = refs[k:k + 5]
        k += 5
        o_ref = refs[k] if want_full else None
        k += 1 if want_full else 0
        gm_ref = refs[k] if want_gmean else None
        k += 1 if want_gmean else 0
        pm_ref = refs[k] if want_pmean else None

        i = pl.program_id(0)
        h = jnp.broadcast_to(b1_ref[0], (block, 128))
        for (a_ref, w_ref) in seg_refs:
            h = h + jnp.dot(a_ref[...], w_ref[...], preferred_element_type=F32)
        if has_extra:
            h = h + extra_ref[...]
        h = jnp.maximum(h, 0.0)
        h = jnp.maximum(jnp.dot(h, W2_ref[...], preferred_element_type=F32)
                        + b2_ref[...], 0.0)
        o = jnp.dot(h, W3_ref[...], preferred_element_type=F32) + b3_ref[...]
        if want_full:
            o_ref[...] = o
        if want_gmean:
            v = jnp.sum(o * gate_ref[...], axis=0, keepdims=True)[None]

            @pl.when(i % bpg == 0)
            def _():
                gm_ref[...] = jnp.zeros_like(gm_ref)
            gm_ref[...] += v

            @pl.when(i % bpg == bpg - 1)
            def _():
                gm_ref[...] *= mean_scale
        if want_pmean:
            v2 = jnp.sum(o, axis=0, keepdims=True)[None]

            @pl.when(i % bpg == 0)
            def _():
                pm_ref[...] = jnp.zeros_like(pm_ref)
            pm_ref[...] += v2

            @pl.when(i % bpg == bpg - 1)
            def _():
                pm_ref[...] *= mean_scale

    res = pl.pallas_call(
        body, grid=(grid,), in_specs=in_specs,
        out_specs=out_specs if len(out_specs) > 1 else out_specs[0],
        out_shape=out_shape if len(out_shape) > 1 else out_shape[0],
    )(*inputs)
    res = list(res) if isinstance(res, (tuple, list)) else [res]
    k = 1 if want_full else 0
    for j in range(k, len(res)):
        res[j] = res[j].reshape(ngr, dout)
    return tuple(res)


def _proj_call(segs, *, block):
    R = segs[0][0].shape[0]
    grid = R // block
    inputs = []
    in_specs = []
    for (a, W) in segs:
        d = a.shape[1]
        inputs.append(a)
        in_specs.append(pl.BlockSpec((block, d), lambda i: (i, 0)))
        inputs.append(W)
        in_specs.append(pl.BlockSpec(W.shape, lambda i: (0, 0)))
    nsegs = len(segs)

    def body(*refs):
        acc = None
        for s in range(nsegs):
            a_ref, w_ref = refs[2 * s], refs[2 * s + 1]
            d = jnp.dot(a_ref[...], w_ref[...], preferred_element_type=F32)
            acc = d if acc is None else acc + d
        refs[-1][...] = acc

    return pl.pallas_call(
        body, grid=(grid,), in_specs=in_specs,
        out_specs=pl.BlockSpec((block, 128), lambda i: (i, 0)),
        out_shape=jax.ShapeDtypeStruct((R, 128), F32),
    )(*inputs)


def _tiny3(x, l1, l2, l3, gate_pair=None):
    r = x.shape[0]
    xp = jnp.pad(x, ((0, 8 - r), (0, 0)))
    (W1, b1), (W2, b2), (W3, b3) = l1, l2, l3
    dout = W3.shape[1]
    inputs = [xp, W1, b1.reshape(1, -1), W2, b2.reshape(1, -1),
              W3, b3.reshape(1, -1)]
    if gate_pair is not None:
        ga = jnp.pad(gate_pair[0], ((0, 8 - r), (0, 0)))
        gb = jnp.pad(gate_pair[1], ((0, 8 - r), (0, 0)))
        inputs += [ga, gb]
    has_gate = gate_pair is not None

    def body(*refs):
        x_ref, W1r, b1r, W2r, b2r, W3r, b3r = refs[:7]
        o_ref = refs[-1]
        xv = x_ref[...]
        if has_gate:
            xv = jnp.concatenate([xv, refs[7][...] * refs[8][...]], axis=1)
        h = jnp.maximum(jnp.dot(xv, W1r[...], preferred_element_type=F32)
                        + b1r[...], 0.0)
        h = jnp.maximum(jnp.dot(h, W2r[...], preferred_element_type=F32)
                        + b2r[...], 0.0)
        o_ref[...] = jnp.dot(h, W3r[...], preferred_element_type=F32) + b3r[...]

    out = pl.pallas_call(
        body, out_shape=jax.ShapeDtypeStruct((8, dout), F32),
    )(*inputs)
    return out[:r]


def _tiny_affine(x, W, b):
    r = x.shape[0]
    xp = jnp.pad(x, ((0, 8 - r), (0, 0)))

    def body(x_ref, w_ref, b_ref, o_ref):
        o_ref[...] = (jnp.dot(x_ref[...], w_ref[...], preferred_element_type=F32)
                      + b_ref[...])

    out = pl.pallas_call(
        body, out_shape=jax.ShapeDtypeStruct((8, W.shape[1]), F32),
    )(xp, W, b.reshape(1, -1))
    return out[:r]


_BQ = 1024
_BK = 2048
_NPAD = 10240


def _flash_cosine(qr, qh, sr, sh):
    grid_i = _NPAD // _BQ
    grid_j = _NPAD // _BK

    def body(qr_ref, qh_ref, sr_ref, sh_ref, o_ref, acc_ref, m_ref, l_ref):
        j = pl.program_id(1)

        @pl.when(j == 0)
        def _():
            acc_ref[...] = jnp.zeros_like(acc_ref)
            m_ref[...] = jnp.full_like(m_ref, -1e30)
            l_ref[...] = jnp.zeros_like(l_ref)

        q = jnp.concatenate([qr_ref[...], qh_ref[...]], axis=1)
        s = jnp.concatenate([sr_ref[...], sh_ref[...]], axis=1)
        qn = q / (jnp.sqrt(jnp.sum(q * q, axis=1, keepdims=True)) + 1e-8)
        sn = s / (jnp.sqrt(jnp.sum(s * s, axis=1, keepdims=True)) + 1e-8)
        logits = lax.dot_general(qn.astype(jnp.bfloat16), sn.astype(jnp.bfloat16),
                                 (((1,), (1,)), ((), ())),
                                 preferred_element_type=F32)
        col = j * _BK + lax.broadcasted_iota(jnp.int32, (_BQ, _BK), 1)
        logits = jnp.where(col < N, logits, -1e30)
        m_old = m_ref[...]
        m_new = jnp.maximum(m_old, jnp.max(logits, axis=1, keepdims=True))
        alpha = jnp.exp(m_old - m_new)
        p = jnp.exp(logits - m_new)
        l_ref[...] = l_ref[...] * alpha + jnp.sum(p, axis=1, keepdims=True)
        acc_ref[...] = (acc_ref[...] * alpha
                        + jnp.dot(p.astype(jnp.bfloat16), s.astype(jnp.bfloat16),
                                  preferred_element_type=F32))
        m_ref[...] = m_new

        @pl.when(j == grid_j - 1)
        def _():
            o_ref[...] = acc_ref[...] / l_ref[...]

    out = pl.pallas_call(
        body,
        grid=(grid_i, grid_j),
        in_specs=[
            pl.BlockSpec((_BQ, 128), lambda i, j: (i, 0)),
            pl.BlockSpec((_BQ, 128), lambda i, j: (i, 0)),
            pl.BlockSpec((_BK, 128), lambda i, j: (j, 0)),
            pl.BlockSpec((_BK, 128), lambda i, j: (j, 0)),
        ],
        out_specs=pl.BlockSpec((_BQ, 256), lambda i, j: (i, 0)),
        out_shape=jax.ShapeDtypeStruct((_NPAD, 256), F32),
        scratch_shapes=[
            pltpu.VMEM((_BQ, 256), F32),
            pltpu.VMEM((_BQ, 1), F32),
            pltpu.VMEM((_BQ, 1), F32),
        ],
    )(qr, qh, sr, sh)
    return out[:N]


def _pad_rows(a):
    return jnp.pad(a, ((0, _NPAD - a.shape[0]), (0, 0)))


def _sc_gather_diff(table_p, dstl, srcl):
    CH = 80
    n_ch = dstl.shape[1]
    per_t = n_ch * CH
    M = 32 * per_t
    rows_t = _NPAD // 16
    mesh = plsc.VectorSubcoreMesh(core_axis_name="c", subcore_axis_name="s")

    @functools.partial(
        pl.kernel, mesh=mesh,
        out_type=jax.ShapeDtypeStruct((M, 128), F32),
        scratch_types=[
            pltpu.VMEM((CH,), jnp.int32),
            pltpu.VMEM((CH,), jnp.int32),
            pltpu.VMEM((CH,), jnp.int32),
            pltpu.VMEM((CH,), jnp.int32),
            pltpu.VMEM((CH, 128), F32),
            pltpu.VMEM((CH, 128), F32),
            pltpu.VMEM((CH, 128), F32),
            pltpu.VMEM((CH, 128), F32),
            pltpu.SemaphoreType.DMA,
            pltpu.SemaphoreType.DMA,
            pltpu.SemaphoreType.DMA,
            pltpu.SemaphoreType.DMA,
            pltpu.SemaphoreType.DMA,
            pltpu.SemaphoreType.DMA,
            pltpu.SemaphoreType.DMA,
            pltpu.SemaphoreType.DMA,
            pltpu.SemaphoreType.DMA,
            pltpu.SemaphoreType.DMA,
        ])
    def k(table_h, dst_h, src_h, out_h,
          idxd0, idxd1, idxs0, idxs1,
          bufd0, bufd1, bufs0, bufs1,
          gd0, gd1, gs0, gs1, so0, so1, id0, id1, is0, is1):
        c = lax.axis_index("c")
        s = lax.axis_index("s")
        w = c * 16 + s
        base = c * (M // 2) + s * per_t

        idxd = (idxd0, idxd1)
        idxs = (idxs0, idxs1)
        bufd = (bufd0, bufd1)
        bufs = (bufs0, bufs1)
        gsd = (gd0, gd1)
        gss = (gs0, gs1)
        osem = (so0, so1)
        isd = (id0, id1)
        iss = (is0, is1)

        def idx_issue(ci, b):
            pltpu.async_copy(dst_h.at[w, ci], idxd[b], isd[b])
            pltpu.async_copy(src_h.at[w, ci], idxs[b], iss[b])

        def wait_i(b):
            pltpu.make_async_copy(dst_h.at[w, 0], idxd[b], isd[b]).wait()
            pltpu.make_async_copy(src_h.at[w, 0], idxs[b], iss[b]).wait()

        def issue(b):
            pltpu.async_copy(table_h.at[idxd[b]], bufd[b], gsd[b])
            pltpu.async_copy(table_h.at[idxs[b]], bufs[b], gss[b])

        def wait_g(b):
            pltpu.make_async_copy(table_h.at[idxd[b]], bufd[b], gsd[b]).wait()
            pltpu.make_async_copy(table_h.at[idxs[b]], bufs[b], gss[b]).wait()

        def sub(b):
            D, S = bufd[b], bufs[b]

            def row(r, c2):
                for l in range(8):
                    sl = pl.ds(l * 16, 16)
                    D[r, sl] = D[r, sl] - S[r, sl]
                return c2
            lax.fori_loop(0, CH, row, 0, unroll=8)

        def out_issue(ci, b):
            pltpu.async_copy(bufd[b], out_h.at[pl.ds(base + ci * CH, CH)],
                             osem[b])

        def wait_out(b):
            pltpu.make_async_copy(bufd[b], out_h.at[pl.ds(base, CH)],
                                  osem[b]).wait()

        idx_issue(0, 0)
        idx_issue(1, 1)
        wait_i(0)
        issue(0)
        wait_g(0); idx_issue(2, 0); wait_i(1); issue(1); sub(0)
        out_issue(0, 0)
        wait_g(1); idx_issue(3, 1); wait_out(0); wait_i(0); issue(0); sub(1)
        out_issue(1, 1)

        def body(t, carry):
            i0 = 2 * t
            wait_g(0); idx_issue(i0 + 2, 0); wait_out(1); wait_i(1); issue(1)
            sub(0); out_issue(i0, 0)
            wait_g(1); idx_issue(i0 + 3, 1); wait_out(0); wait_i(0); issue(0)
            sub(1); out_issue(i0 + 1, 1)
            return carry
        lax.fori_loop(1, n_ch // 2 - 1, body, 0)

        wait_g(0); wait_out(1); wait_i(1); issue(1); sub(0)
        out_issue(n_ch - 2, 0)
        wait_g(1); wait_out(0); sub(1); out_issue(n_ch - 1, 1)
        wait_out(1)

    return k(table_p, dstl, srcl)


def _sc_scatter_add(vals, dst_local, zeros):
    per_t = E // 16
    CH = 80
    n_ch = per_t // CH
    rows_t = _NPAD // 16
    mesh = plsc.VectorSubcoreMesh(core_axis_name="c", subcore_axis_name="s")

    @functools.partial(
        pl.kernel, mesh=mesh,
        out_type=jax.ShapeDtypeStruct((2, _NPAD, 128), F32),
        scratch_types=[
            pltpu.VMEM((CH,), jnp.int32),
            pltpu.VMEM((CH,), jnp.int32),
            pltpu.VMEM((CH, 128), F32),
            pltpu.VMEM((CH, 128), F32),
            pltpu.VMEM_SHARED((_NPAD, 128), F32),
            pltpu.SemaphoreType.DMA,
            pltpu.SemaphoreType.DMA,
            pltpu.SemaphoreType.DMA,
            pltpu.SemaphoreType.DMA,
        ])
    def k(vals_h, dst_h, zeros_h, out_h, idx0, idx1, vbuf0, vbuf1, acc,
          vs0, vs1, is0, is1):
        c = lax.axis_index("c")
        s = lax.axis_index("s")
        w = c * 16 + s
        for j in range(rows_t // CH):
            sl = pl.ds(s * rows_t + j * CH, CH)
            pltpu.sync_copy(zeros_h.at[sl], vbuf0)
            pltpu.sync_copy(vbuf0, acc.at[sl])
        plsc.subcore_barrier()
        base = c * E + s * per_t

        idxb = (idx0, idx1)
        isem = (is0, is1)
        vbuf = (vbuf0, vbuf1)
        vsem = (vs0, vs1)

        def idx_issue(ci, b):
            pltpu.async_copy(dst_h.at[w, ci], idxb[b], isem[b])

        def wait_i(b):
            pltpu.make_async_copy(dst_h.at[w, 0], idxb[b], isem[b]).wait()

        def load_issue(ci, b):
            pltpu.async_copy(vals_h.at[pl.ds(base + ci * CH, CH)], vbuf[b],
                             vsem[b])

        def wait_v(b):
            pltpu.make_async_copy(vals_h.at[pl.ds(base, CH)], vbuf[b],
                                  vsem[b]).wait()

        def scat(b):
            pltpu.sync_copy(vbuf[b], acc.at[idxb[b]], add=True)

        idx_issue(0, 0)
        idx_issue(1, 1)
        load_issue(0, 0)

        def body(t, carry):
            i0 = 2 * t
            load_issue(i0 + 1, 1); wait_v(0); wait_i(0); scat(0)
            idx_issue(i0 + 2, 0)
            load_issue(i0 + 2, 0); wait_v(1); wait_i(1); scat(1)
            idx_issue(i0 + 3, 1)
            return carry
        lax.fori_loop(0, n_ch // 2 - 1, body, 0)

        load_issue(n_ch - 1, 1); wait_v(0); wait_i(0); scat(0)
        wait_v(1); wait_i(1); scat(1)

        plsc.subcore_barrier()
        for j in range(rows_t // CH):
            sl = pl.ds(s * rows_t + j * CH, CH)
            pltpu.sync_copy(acc.at[sl], vbuf0)
            pltpu.sync_copy(vbuf0, out_h.at[c, sl])

    return k(vals, dst_local, zeros)[:, :N]


def kernel(x1, edge_index1, e1, u1, x2, edge_index2, e2, u2, params):
    p = params

    x12 = jnp.concatenate([x1, x2], 0)
    e12 = jnp.concatenate([e1, e2], 0)
    src_l = jnp.concatenate([edge_index1[0], edge_index2[0]])
    dst_l = jnp.concatenate([edge_index1[1], edge_index2[1]])
    src3 = src_l.reshape(32, 250, 80)
    dst3 = dst_l.reshape(32, 250, 80)
    src3g = jnp.concatenate(
        [edge_index1[0], edge_index2[0] + _NPAD]).reshape(32, 250, 80)
    dst3g = jnp.concatenate(
        [edge_index1[1], edge_index2[1] + _NPAD]).reshape(32, 250, 80)
    zeros_n = jnp.zeros((_NPAD, 128), F32)
    zpad = jnp.zeros((_NPAD - N, 128), F32)

    def _pad_table(t):
        return jnp.concatenate([t[:N], zpad, t[N:], zpad], 0)

    (enW1, enb1), (enW2, enb2), (enW3, enb3) = p['enc_node']
    (xh12,) = _mlp3_call([(x12, enW1)], None, enb1.reshape(1, -1),
                         enW2, enb2.reshape(1, -1), enW3, enb3.reshape(1, -1),
                         block=800, bpg=25)
    (eeW1, eeb1), (eeW2, eeb2), (eeW3, eeb3) = p['enc_edge']
    (eh12,) = _mlp3_call([(e12, eeW1)], None, eeb1.reshape(1, -1),
                         eeW2, eeb2.reshape(1, -1), eeW3, eeb3.reshape(1, -1),
                         block=8000, bpg=80)
    uh = _tiny3(jnp.stack([u1, u2]), *p['enc_glob'])
    u_cat = jnp.concatenate([jnp.stack([u1, u2]), uh], 1)

    x1h, x2h = xh12[:N], xh12[N:]

    (W1, b1), (W2, b2), (W3, b3) = p['rec_edge']
    W1x, W1er, W1eh, W1u = W1[0:256], W1[256:272], W1[272:400], W1[400:656]
    Pt = _proj_call([(x12, W1x[:128]), (xh12, W1x[128:])], block=800)
    G = _sc_gather_diff(_pad_table(Pt), dst3g, src3g)
    bias_re = _tiny_affine(u_cat, W1u, b1)
    e_new, em = _mlp3_call(
        [(e12, W1er), (eh12, W1eh)], G, bias_re,
        W2, b2.reshape(1, -1), W3, b3.reshape(1, -1),
        block=8000, bpg=40, want_pmean=True, mean_scale=1.0 / E)
    agg = _sc_scatter_add(e_new, dst3, zeros_n)

    (N1, nb1), (N2, nb2), (N3, nb3) = p['rec_node']
    NWa, NWx, NWt, NWu = N1[0:128], N1[128:384], N1[384:640], N1[640:896]
    bias_rn = _tiny_affine(u_cat, NWu, nb1)

    att1 = _flash_cosine(_pad_rows(x1), _pad_rows(x1h),
                         _pad_rows(x2), _pad_rows(x2h))
    x1n, xm1 = _mlp3_call(
        [(agg[0], NWa), (x1, NWx[:128]), (x1h, NWx[128:]), (att1, NWt)],
        None, bias_rn[0:1], N2, nb2.reshape(1, -1), N3, nb3.reshape(1, -1),
        block=2000, bpg=5, want_pmean=True, mean_scale=1.0 / N)
    u1n = _tiny3(jnp.concatenate([xm1, em[0:1], u_cat[0:1]], 1), *p['rec_glob'])

    att2 = _flash_cosine(_pad_rows(x2), _pad_rows(x2h),
                         _pad_rows(x1), _pad_rows(x1n))
    x2n, xm2 = _mlp3_call(
        [(agg[1], NWa), (x2, NWx[:128]), (x2h, NWx[128:]), (att2, NWt)],
        None, bias_rn[1:2], N2, nb2.reshape(1, -1), N3, nb3.reshape(1, -1),
        block=2000, bpg=5, want_pmean=True, mean_scale=1.0 / N)
    u2n = _tiny3(jnp.concatenate([xm2, em[1:2], u_cat[1:2]], 1), *p['rec_glob'])

    un = jnp.concatenate([u1n, u2n], 0)
    xn = jnp.concatenate([x1n, x2n], 0)

    (A1, ab1), (A2, ab2), (A3, ab3) = p['att_edge']
    A1x, A1e, A1u = A1[0:128], A1[128:256], A1[256:384]
    Pa = _proj_call([(xn, A1x)], block=800)
    Ga = _sc_gather_diff(_pad_table(Pa), dst3g, src3g)
    bias_ae = _tiny_affine(un, A1u, ab1)
    ea, egm, eam = _mlp3_call(
        [(e_new, A1e)], Ga, bias_ae,
        A2, ab2.reshape(1, -1), A3, ab3.reshape(1, -1),
        block=8000, bpg=40, gate=e_new, want_gmean=True, want_pmean=True,
        mean_scale=1.0 / E)
    agga = _sc_scatter_add(ea, dst3, zeros_n)

    (B1, bb1), (B2, bb2), (B3, bb3) = p['att_node']
    B1a, B1x, B1u = B1[0:128], B1[128:256], B1[256:384]
    biasn = _tiny_affine(un, B1u, bb1)
    xgm, xam = _mlp3_call(
        [(agga.reshape(2 * N, 128), B1a), (xn, B1x)], None, biasn,
        B2, bb2.reshape(1, -1), B3, bb3.reshape(1, -1),
        block=2000, bpg=5, gate=xn, want_full=False, want_gmean=True,
        want_pmean=True, mean_scale=1.0 / N)

    ua = _tiny3(jnp.concatenate([xam, eam, un], 1), *p['att_glob'])
    uf = _tiny3(jnp.concatenate([xgm, egm], 1), *p['agg_glob'],
                gate_pair=(un, ua))
    out = _tiny3(uf.reshape(1, 256), *p['final'])
    return out.reshape(64)
